# Initial kernel scaffold; baseline (speedup 1.0000x reference)
import jax
import jax.numpy as jnp
from jax.experimental import pallas as pl


def kernel(_input, dependency_triples, W, a):
    raise NotImplementedError("write your pallas kernel here")


# speedup vs baseline: 2.5246x; 2.5246x over previous
import functools

import jax
import jax.numpy as jnp
from jax import lax
from jax.experimental import pallas as pl
from jax.experimental.pallas import tpu as pltpu
from jax.experimental.pallas import tpu_sc as plsc

_L = 16
_NW = 32
_CH = 64


def _tc_mm(x_ref, w_ref, a2_ref, wx_ref, sgd_ref):
    wx = lax.dot_general(x_ref[...], w_ref[...], (((1,), (1,)), ((), ())),
                         preferred_element_type=jnp.float32)
    wx_ref[...] = wx
    sgd_ref[...] = lax.dot_general(wx, a2_ref[...], (((1,), (1,)), ((), ())),
                                   preferred_element_type=jnp.float32)


def _sc_kernel(n, m, d, rpw,
               wx_hbm, sg_hbm, sd_hbm, sgov_hbm, sdep_hbm, winner_hbm,
               rs_hbm, cnt_hbm, h0gov_hbm, outpos_hbm, out_hbm,
               sgov_v, sdep_v, winner_v, score_v, val_v, sg_v, sd_v,
               rs_v, cnt_v, h0gov_v, outpos_v, gbuf, agg_v, hbuf, sem):
    c_id = lax.axis_index("c")
    s_id = lax.axis_index("s")
    wid = s_id * 2 + c_id
    r0 = wid * rpw

    def sload(ref, i):
        return ref[pl.ds(i, _L)][0]

    pltpu.sync_copy(sgov_hbm, sgov_v.at[pl.ds(0, m)])
    pltpu.sync_copy(sdep_hbm, sdep_v)
    pltpu.sync_copy(winner_hbm, winner_v)
    pltpu.sync_copy(sg_hbm, sg_v)
    pltpu.sync_copy(sd_hbm, sd_v)
    pltpu.sync_copy(rs_hbm.at[pl.ds(r0, rpw)], rs_v.at[pl.ds(0, rpw)])
    pltpu.sync_copy(cnt_hbm.at[pl.ds(r0, rpw)], cnt_v.at[pl.ds(0, rpw)])
    pltpu.sync_copy(h0gov_hbm.at[pl.ds(r0, rpw)], h0gov_v)
    pltpu.sync_copy(outpos_hbm.at[pl.ds(r0, rpw)], outpos_v)

    iota = lax.iota(jnp.int32, _L)
    NEG = jnp.float32(-3.0e38)

    def score_body(i, _):
        b = i * _L
        gi = sgov_v[pl.ds(b, _L)]
        di = sdep_v[pl.ds(b, _L)]
        score_v[pl.ds(b, _L)] = (plsc.load_gather(sg_v, [gi]) +
                                 plsc.load_gather(sd_v, [di]))
        return 0

    lax.fori_loop(0, m // _L, score_body, 0)

    def val_body(i, _):
        b = i * _L
        wv = winner_v[pl.ds(b, _L)]
        val_v[pl.ds(b, _L)] = plsc.load_gather(score_v, [wv])
        return 0

    lax.fori_loop(0, m // _L, val_body, 0)

    def row_body(r, _):
        s = sload(rs_v, r)
        c = sload(cnt_v, r)
        e_end = s + c
        b0 = (s // _L) * _L
        nch = (e_end - b0 + _L - 1) // _L

        def max_body(k, mc):
            b = b0 + k * _L
            v = val_v[pl.ds(b, _L)]
            g = b + iota
            msk = (g >= s) & (g < e_end) & (v > 0)
            return jnp.maximum(mc, jnp.max(jnp.where(msk, v, NEG), axis=0))

        def soft_iter(it, mc):
            def sum_body(k, acc):
                b = b0 + k * _L
                v = val_v[pl.ds(b, _L)]
                w = winner_v[pl.ds(b, _L)]
                g = b + iota
                msk = (g >= s) & (g < e_end) & (v > 0) & (w == g)
                return acc + jnp.sum(jnp.where(msk, jnp.exp(v - mc), 0.0),
                                     axis=0)

            den = lax.fori_loop(0, nch, sum_body, jnp.float32(0.0))

            def upd_body(k, _):
                b = b0 + k * _L
                v = val_v[pl.ds(b, _L)]
                g = b + iota
                rm = (g >= s) & (g < e_end)
                pos = rm & (v > 0)
                val_v[pl.ds(b, _L)] = jnp.where(
                    pos, jnp.exp(v - mc) / den, jnp.where(rm, 0.0, v))
                return 0

            lax.fori_loop(0, nch, upd_body, 0)
            return (jnp.full((_L,), 1.0, jnp.float32) / den)[0]

        def do_rows(_):
            m0 = lax.fori_loop(0, nch, max_body, NEG)

            def do_soft(_):
                lax.fori_loop(0, c, soft_iter, m0)
                return 0

            def do_unif(_):
                u = jnp.float32(1.0 / n)

                def ub(k, _):
                    b = b0 + k * _L
                    v = val_v[pl.ds(b, _L)]
                    g = b + iota
                    rm = (g >= s) & (g < e_end)
                    val_v[pl.ds(b, _L)] = jnp.where(rm, u, v)
                    return 0

                lax.fori_loop(0, nch, ub, 0)
                return 0

            lax.cond(m0 > NEG, do_soft, do_unif, 0)
            return 0

        lax.cond(c > 0, do_rows, lambda _: 0, 0)
        return 0

    lax.fori_loop(0, rpw, row_body, 0)

    zero16 = jnp.zeros((_L,), jnp.float32)

    def z_body(i, _):
        for j in range(d // _L):
            agg_v[i, pl.ds(j * _L, _L)] = zero16
        return 0

    lax.fori_loop(0, rpw, z_body, 0)

    s0 = sload(rs_v, 0)
    s1 = sload(rs_v, rpw - 1) + sload(cnt_v, rpw - 1)
    k0 = s0 // _CH
    k1 = (s1 + _CH - 1) // _CH

    def chunk_body(k, _):
        b = k * _CH
        pltpu.async_copy(wx_hbm.at[sdep_v.at[pl.ds(b, _CH)]], gbuf, sem).wait()

        def e_body(e, _):
            p = b + e

            def acc(_):
                row = sload(sgov_v, p) - r0
                av = sload(val_v, p)
                for j in range(d // _L):
                    cur = agg_v[row, pl.ds(j * _L, _L)]
                    agg_v[row, pl.ds(j * _L, _L)] = (
                        cur + av * gbuf[e, pl.ds(j * _L, _L)])
                return 0

            lax.cond((p >= s0) & (p < s1), acc, lambda _: 0, 0)
            return 0

        lax.fori_loop(0, _CH, e_body, 0)
        return 0

    lax.fori_loop(k0, k1, chunk_body, 0)

    pltpu.async_copy(wx_hbm.at[h0gov_v], hbuf, sem).wait()

    def o_body(i, _):
        for j in range(d // _L):
            t = agg_v[i, pl.ds(j * _L, _L)] + hbuf[i, pl.ds(j * _L, _L)]
            hbuf[i, pl.ds(j * _L, _L)] = jnp.where(t >= 0, t, 0.2 * t)
        return 0

    lax.fori_loop(0, rpw, o_body, 0)
    pltpu.async_copy(hbuf, out_hbm.at[outpos_v], sem).wait()


def kernel(_input, dependency_triples, W, a):
    n, d = _input.shape
    m = dependency_triples.shape[0]
    rpw = n // _NW

    deps = dependency_triples[:, 0].astype(jnp.int32)
    govs = dependency_triples[:, 2].astype(jnp.int32)
    idx = jnp.arange(m, dtype=jnp.int32)

    ord1 = jnp.argsort(govs, stable=True)
    sgov = govs[ord1]
    sdep = deps[ord1]
    rank1 = jnp.zeros((m,), jnp.int32).at[ord1].set(idx)

    kcell = govs * n + deps
    ord2 = jnp.argsort(kcell, stable=True)
    kc_s = kcell[ord2]
    is_last_s = jnp.concatenate(
        [kc_s[1:] != kc_s[:-1], jnp.ones((1,), bool)])
    t = jnp.where(is_last_s, idx, m)
    wpos = jnp.flip(lax.cummin(jnp.flip(t)))
    winner_edge = jnp.zeros((m,), jnp.int32).at[ord2].set(ord2[wpos])
    winner_pos = rank1[winner_edge[ord1]]

    counts = jnp.bincount(govs, length=n).astype(jnp.int32)
    row_start = (jnp.cumsum(counts) - counts).astype(jnp.int32)

    last_idx = jnp.full((n,), -1, jnp.int32).at[deps].max(idx)
    h0gov = govs[last_idx]
    first_idx = jnp.full((n,), m, jnp.int32).at[deps].min(idx)
    first_idx = first_idx.at[0].set(-1)
    keys = jnp.argsort(first_idx)
    outpos = jnp.zeros((n,), jnp.int32).at[keys].set(jnp.arange(n, dtype=jnp.int32))

    a2 = a.reshape(2, d)
    wx, sgd = pl.pallas_call(
        _tc_mm,
        out_shape=[jax.ShapeDtypeStruct((n, d), jnp.float32),
                   jax.ShapeDtypeStruct((n, 2), jnp.float32)],
    )(_input, W, a2)
    sg = sgd[:, 0]
    sd = sgd[:, 1]

    mesh = plsc.VectorSubcoreMesh(core_axis_name="c", subcore_axis_name="s")
    sc = pl.kernel(
        functools.partial(_sc_kernel, n, m, d, rpw),
        mesh=mesh,
        compiler_params=pltpu.CompilerParams(needs_layout_passes=False),
        out_type=jax.ShapeDtypeStruct((n, d), jnp.float32),
        scratch_types=[
            pltpu.VMEM((m + _L,), jnp.int32),
            pltpu.VMEM((m,), jnp.int32),
            pltpu.VMEM((m,), jnp.int32),
            pltpu.VMEM((m,), jnp.float32),
            pltpu.VMEM((m + _L,), jnp.float32),
            pltpu.VMEM((n,), jnp.float32),
            pltpu.VMEM((n,), jnp.float32),
            pltpu.VMEM((rpw + _L,), jnp.int32),
            pltpu.VMEM((rpw + _L,), jnp.int32),
            pltpu.VMEM((rpw,), jnp.int32),
            pltpu.VMEM((rpw,), jnp.int32),
            pltpu.VMEM((_CH, d), jnp.float32),
            pltpu.VMEM((rpw, d), jnp.float32),
            pltpu.VMEM((rpw, d), jnp.float32),
            pltpu.SemaphoreType.DMA,
        ],
    )
    return sc(wx, sg, sd, sgov, sdep, winner_pos,
              row_start, counts, h0gov, outpos)


# speedup vs baseline: 2.6183x; 1.0371x over previous
import functools

import jax
import jax.numpy as jnp
from jax import lax
from jax.experimental import pallas as pl
from jax.experimental.pallas import tpu as pltpu
from jax.experimental.pallas import tpu_sc as plsc

_L = 16
_NW = 32
_CH = 64
_EC = 512


def _tc_mm(x_ref, w_ref, a2_ref, wx_ref, sgd_ref):
    wx = lax.dot_general(x_ref[...], w_ref[...], (((1,), (1,)), ((), ())),
                         preferred_element_type=jnp.float32)
    wx_ref[...] = wx
    sgd_ref[...] = lax.dot_general(wx, a2_ref[...], (((1,), (1,)), ((), ())),
                                   preferred_element_type=jnp.float32)


def _sc_kernel(n, m, d, rpw,
               wx_hbm, sg_hbm, sd_hbm, sgov_hbm, sdep_hbm, winner_hbm,
               rs_hbm, cnt_hbm, h0gov_hbm, outpos_hbm, out_hbm,
               sgov_v, sdep_v, winner_v, score_v, val_v, sg_v, sd_v,
               rs_v, cnt_v, h0gov_v, outpos_v, gbuf, agg_v, hbuf, sem):
    c_id = lax.axis_index("c")
    s_id = lax.axis_index("s")
    wid = s_id * 2 + c_id
    r0 = wid * rpw

    def sload(ref, i):
        return ref[pl.ds(i, _L)][0]

    pltpu.sync_copy(rs_hbm.at[pl.ds(r0, rpw)], rs_v.at[pl.ds(0, rpw)])
    pltpu.sync_copy(cnt_hbm.at[pl.ds(r0, rpw)], cnt_v.at[pl.ds(0, rpw)])
    pltpu.sync_copy(h0gov_hbm.at[pl.ds(r0, rpw)], h0gov_v)
    pltpu.sync_copy(outpos_hbm.at[pl.ds(r0, rpw)], outpos_v)
    pltpu.sync_copy(sg_hbm.at[pl.ds(r0, rpw)], sg_v.at[pl.ds(r0, rpw)])
    pltpu.sync_copy(sd_hbm, sd_v)

    s0 = sload(rs_v, 0)
    s1 = sload(rs_v, rpw - 1) + sload(cnt_v, rpw - 1)
    b0a = (s0 // _CH) * _CH
    nblk = (s1 - b0a + _L - 1) // _L

    def edma_body(k, _):
        b = b0a + k * _EC
        pltpu.sync_copy(sgov_hbm.at[pl.ds(b, _EC)], sgov_v.at[pl.ds(b, _EC)])
        pltpu.sync_copy(sdep_hbm.at[pl.ds(b, _EC)], sdep_v.at[pl.ds(b, _EC)])
        pltpu.sync_copy(winner_hbm.at[pl.ds(b, _EC)],
                        winner_v.at[pl.ds(b, _EC)])
        return 0

    lax.fori_loop(0, (s1 - b0a + _EC - 1) // _EC, edma_body, 0)

    iota = lax.iota(jnp.int32, _L)
    NEG = jnp.float32(-3.0e38)

    def score_body(i, _):
        b = b0a + i * _L
        gi = sgov_v[pl.ds(b, _L)]
        di = sdep_v[pl.ds(b, _L)]
        score_v[pl.ds(b, _L)] = (plsc.load_gather(sg_v, [gi]) +
                                 plsc.load_gather(sd_v, [di]))
        return 0

    lax.fori_loop(0, nblk, score_body, 0)

    def val_body(i, _):
        b = b0a + i * _L
        wv = winner_v[pl.ds(b, _L)]
        val_v[pl.ds(b, _L)] = plsc.load_gather(score_v, [wv])
        return 0

    lax.fori_loop(0, nblk, val_body, 0)

    def row_body(r, _):
        s = sload(rs_v, r)
        c = sload(cnt_v, r)
        e_end = s + c
        b0 = (s // _L) * _L
        nch = (e_end - b0 + _L - 1) // _L

        def max_body(k, mc):
            b = b0 + k * _L
            v = val_v[pl.ds(b, _L)]
            g = b + iota
            msk = (g >= s) & (g < e_end) & (v > 0)
            return jnp.maximum(mc, jnp.max(jnp.where(msk, v, NEG), axis=0))

        def soft_iter(it, mc):
            def sum_body(k, acc):
                b = b0 + k * _L
                v = val_v[pl.ds(b, _L)]
                w = winner_v[pl.ds(b, _L)]
                g = b + iota
                msk = (g >= s) & (g < e_end) & (v > 0) & (w == g)
                return acc + jnp.sum(jnp.where(msk, jnp.exp(v - mc), 0.0),
                                     axis=0)

            den = lax.fori_loop(0, nch, sum_body, jnp.float32(0.0))

            def upd_body(k, _):
                b = b0 + k * _L
                v = val_v[pl.ds(b, _L)]
                g = b + iota
                rm = (g >= s) & (g < e_end)
                pos = rm & (v > 0)
                val_v[pl.ds(b, _L)] = jnp.where(
                    pos, jnp.exp(v - mc) / den, jnp.where(rm, 0.0, v))
                return 0

            lax.fori_loop(0, nch, upd_body, 0)
            return (jnp.full((_L,), 1.0, jnp.float32) / den)[0]

        def do_rows(_):
            m0 = lax.fori_loop(0, nch, max_body, NEG)

            def do_soft(_):
                lax.fori_loop(0, c, soft_iter, m0)
                return 0

            def do_unif(_):
                u = jnp.float32(1.0 / n)

                def ub(k, _):
                    b = b0 + k * _L
                    v = val_v[pl.ds(b, _L)]
                    g = b + iota
                    rm = (g >= s) & (g < e_end)
                    val_v[pl.ds(b, _L)] = jnp.where(rm, u, v)
                    return 0

                lax.fori_loop(0, nch, ub, 0)
                return 0

            lax.cond(m0 > NEG, do_soft, do_unif, 0)
            return 0

        lax.cond(c > 0, do_rows, lambda _: 0, 0)
        return 0

    lax.fori_loop(0, rpw, row_body, 0)

    zero16 = jnp.zeros((_L,), jnp.float32)

    def z_body(i, _):
        for j in range(d // _L):
            agg_v[i, pl.ds(j * _L, _L)] = zero16
        return 0

    lax.fori_loop(0, rpw, z_body, 0)

    k0 = s0 // _CH
    k1 = (s1 + _CH - 1) // _CH

    def chunk_body(k, _):
        b = k * _CH
        pltpu.async_copy(wx_hbm.at[sdep_v.at[pl.ds(b, _CH)]], gbuf, sem).wait()

        def e_body(e, _):
            p = b + e

            def acc(_):
                row = sload(sgov_v, p) - r0
                av = sload(val_v, p)
                for j in range(d // _L):
                    cur = agg_v[row, pl.ds(j * _L, _L)]
                    agg_v[row, pl.ds(j * _L, _L)] = (
                        cur + av * gbuf[e, pl.ds(j * _L, _L)])
                return 0

            lax.cond((p >= s0) & (p < s1), acc, lambda _: 0, 0)
            return 0

        lax.fori_loop(0, _CH, e_body, 0)
        return 0

    lax.fori_loop(k0, k1, chunk_body, 0)

    pltpu.async_copy(wx_hbm.at[h0gov_v], hbuf, sem).wait()

    def o_body(i, _):
        for j in range(d // _L):
            t = agg_v[i, pl.ds(j * _L, _L)] + hbuf[i, pl.ds(j * _L, _L)]
            hbuf[i, pl.ds(j * _L, _L)] = jnp.where(t >= 0, t, 0.2 * t)
        return 0

    lax.fori_loop(0, rpw, o_body, 0)
    pltpu.async_copy(hbuf, out_hbm.at[outpos_v], sem).wait()


def kernel(_input, dependency_triples, W, a):
    n, d = _input.shape
    m = dependency_triples.shape[0]
    rpw = n // _NW

    deps = dependency_triples[:, 0].astype(jnp.int32)
    govs = dependency_triples[:, 2].astype(jnp.int32)
    idx = jnp.arange(m, dtype=jnp.int32)

    ord1 = jnp.argsort(govs, stable=True)
    sgov = govs[ord1]
    sdep = deps[ord1]
    rank1 = jnp.zeros((m,), jnp.int32).at[ord1].set(idx)

    kcell = govs * n + deps
    ord2 = jnp.argsort(kcell, stable=True)
    kc_s = kcell[ord2]
    is_last_s = jnp.concatenate(
        [kc_s[1:] != kc_s[:-1], jnp.ones((1,), bool)])
    t = jnp.where(is_last_s, idx, m)
    wpos = jnp.flip(lax.cummin(jnp.flip(t)))
    winner_edge = jnp.zeros((m,), jnp.int32).at[ord2].set(ord2[wpos])
    winner_pos = rank1[winner_edge[ord1]]

    counts = jnp.bincount(govs, length=n).astype(jnp.int32)
    row_start = (jnp.cumsum(counts) - counts).astype(jnp.int32)

    last_idx = jnp.full((n,), -1, jnp.int32).at[deps].max(idx)
    h0gov = govs[last_idx]
    first_idx = jnp.full((n,), m, jnp.int32).at[deps].min(idx)
    first_idx = first_idx.at[0].set(-1)
    keys = jnp.argsort(first_idx)
    outpos = jnp.zeros((n,), jnp.int32).at[keys].set(jnp.arange(n, dtype=jnp.int32))

    a2 = a.reshape(2, d)
    wx, sgd = pl.pallas_call(
        _tc_mm,
        out_shape=[jax.ShapeDtypeStruct((n, d), jnp.float32),
                   jax.ShapeDtypeStruct((n, 2), jnp.float32)],
    )(_input, W, a2)
    sg = sgd[:, 0]
    sd = sgd[:, 1]

    mesh = plsc.VectorSubcoreMesh(core_axis_name="c", subcore_axis_name="s")
    sc = pl.kernel(
        functools.partial(_sc_kernel, n, m, d, rpw),
        mesh=mesh,
        compiler_params=pltpu.CompilerParams(needs_layout_passes=False),
        out_type=jax.ShapeDtypeStruct((n, d), jnp.float32),
        scratch_types=[
            pltpu.VMEM((m + _EC,), jnp.int32),
            pltpu.VMEM((m + _EC,), jnp.int32),
            pltpu.VMEM((m + _EC,), jnp.int32),
            pltpu.VMEM((m + _L,), jnp.float32),
            pltpu.VMEM((m + _L,), jnp.float32),
            pltpu.VMEM((n,), jnp.float32),
            pltpu.VMEM((n,), jnp.float32),
            pltpu.VMEM((rpw + _L,), jnp.int32),
            pltpu.VMEM((rpw + _L,), jnp.int32),
            pltpu.VMEM((rpw,), jnp.int32),
            pltpu.VMEM((rpw,), jnp.int32),
            pltpu.VMEM((_CH, d), jnp.float32),
            pltpu.VMEM((rpw, d), jnp.float32),
            pltpu.VMEM((rpw, d), jnp.float32),
            pltpu.SemaphoreType.DMA,
        ],
    )
    pad = jnp.zeros((_EC,), jnp.int32)
    return sc(wx, sg, sd,
              jnp.concatenate([sgov, pad]),
              jnp.concatenate([sdep, pad]),
              jnp.concatenate([winner_pos, pad]),
              row_start, counts, h0gov, outpos)


# speedup vs baseline: 2.9584x; 1.1299x over previous
import functools

import jax
import jax.numpy as jnp
from jax import lax
from jax.experimental import pallas as pl
from jax.experimental.pallas import tpu as pltpu
from jax.experimental.pallas import tpu_sc as plsc

_L = 16
_NW = 32
_CH = 64
_EC = 512


def _tc_mm(x_ref, w_ref, a2_ref, wx_ref, sgd_ref):
    wx = lax.dot_general(x_ref[...], w_ref[...], (((1,), (1,)), ((), ())),
                         preferred_element_type=jnp.float32)
    wx_ref[...] = wx
    sgd_ref[...] = lax.dot_general(wx, a2_ref[...], (((1,), (1,)), ((), ())),
                                   preferred_element_type=jnp.float32)


def _sc_kernel(n, m, d, rpw,
               wx_hbm, sg_hbm, sd_hbm, sgov_hbm, sdep_hbm, winner_hbm,
               rs_hbm, cnt_hbm, h0gov_hbm, outpos_hbm, out_hbm,
               sgov_v, sdep_v, winner_v, score_v, val_v, sg_v, sd_v,
               rs_v, cnt_v, h0gov_v, outpos_v, gbuf, agg_v, hbuf, sem):
    c_id = lax.axis_index("c")
    s_id = lax.axis_index("s")
    wid = s_id * 2 + c_id
    r0 = wid * rpw

    def sload(ref, i):
        return ref[pl.ds(i, _L)][0]

    pltpu.sync_copy(rs_hbm.at[pl.ds(r0, rpw)], rs_v.at[pl.ds(0, rpw)])
    pltpu.sync_copy(cnt_hbm.at[pl.ds(r0, rpw)], cnt_v.at[pl.ds(0, rpw)])
    pltpu.sync_copy(h0gov_hbm.at[pl.ds(r0, rpw)], h0gov_v)
    pltpu.sync_copy(outpos_hbm.at[pl.ds(r0, rpw)], outpos_v)
    pltpu.sync_copy(sg_hbm.at[pl.ds(r0, rpw)], sg_v.at[pl.ds(r0, rpw)])
    pltpu.sync_copy(sd_hbm, sd_v)

    s0 = sload(rs_v, 0)
    s1 = sload(rs_v, rpw - 1) + sload(cnt_v, rpw - 1)
    b0a = (s0 // _CH) * _CH
    nblk = (s1 - b0a + _L - 1) // _L

    def edma_body(k, _):
        b = b0a + k * _EC
        pltpu.sync_copy(sgov_hbm.at[pl.ds(b, _EC)], sgov_v.at[pl.ds(b, _EC)])
        pltpu.sync_copy(sdep_hbm.at[pl.ds(b, _EC)], sdep_v.at[pl.ds(b, _EC)])
        pltpu.sync_copy(winner_hbm.at[pl.ds(b, _EC)],
                        winner_v.at[pl.ds(b, _EC)])
        return 0

    lax.fori_loop(0, (s1 - b0a + _EC - 1) // _EC, edma_body, 0)

    iota = lax.iota(jnp.int32, _L)
    NEG = jnp.float32(-3.0e38)

    def score_body(i, _):
        b = b0a + i * _L
        gi = sgov_v[pl.ds(b, _L)]
        di = sdep_v[pl.ds(b, _L)]
        score_v[pl.ds(b, _L)] = (plsc.load_gather(sg_v, [gi]) +
                                 plsc.load_gather(sd_v, [di]))
        return 0

    lax.fori_loop(0, nblk, score_body, 0)

    def val_body(i, _):
        b = b0a + i * _L
        wv = winner_v[pl.ds(b, _L)]
        val_v[pl.ds(b, _L)] = plsc.load_gather(score_v, [wv])
        return 0

    lax.fori_loop(0, nblk, val_body, 0)

    def row_body(r, _):
        s = sload(rs_v, r)
        c = sload(cnt_v, r)
        e_end = s + c
        b0 = (s // _L) * _L
        nch = (e_end - b0 + _L - 1) // _L

        def max_body(k, mc):
            b = b0 + k * _L
            v = val_v[pl.ds(b, _L)]
            g = b + iota
            msk = (g >= s) & (g < e_end) & (v > 0)
            return jnp.maximum(mc, jnp.max(jnp.where(msk, v, NEG), axis=0))

        def soft_iter(it, mc):
            def sum_body(k, acc):
                b = b0 + k * _L
                v = val_v[pl.ds(b, _L)]
                w = winner_v[pl.ds(b, _L)]
                g = b + iota
                msk = (g >= s) & (g < e_end) & (v > 0) & (w == g)
                return acc + jnp.sum(jnp.where(msk, jnp.exp(v - mc), 0.0),
                                     axis=0)

            den = lax.fori_loop(0, nch, sum_body, jnp.float32(0.0))

            def upd_body(k, _):
                b = b0 + k * _L
                v = val_v[pl.ds(b, _L)]
                g = b + iota
                rm = (g >= s) & (g < e_end)
                pos = rm & (v > 0)
                val_v[pl.ds(b, _L)] = jnp.where(
                    pos, jnp.exp(v - mc) / den, jnp.where(rm, 0.0, v))
                return 0

            lax.fori_loop(0, nch, upd_body, 0)
            return (jnp.full((_L,), 1.0, jnp.float32) / den)[0]

        def do_rows(_):
            m0 = lax.fori_loop(0, nch, max_body, NEG)

            def do_soft(_):
                lax.fori_loop(0, c, soft_iter, m0)
                return 0

            def do_unif(_):
                u = jnp.float32(1.0 / n)

                def ub(k, _):
                    b = b0 + k * _L
                    v = val_v[pl.ds(b, _L)]
                    g = b + iota
                    rm = (g >= s) & (g < e_end)
                    val_v[pl.ds(b, _L)] = jnp.where(rm, u, v)
                    return 0

                lax.fori_loop(0, nch, ub, 0)
                return 0

            lax.cond(m0 > NEG, do_soft, do_unif, 0)
            return 0

        lax.cond(c > 0, do_rows, lambda _: 0, 0)
        return 0

    lax.fori_loop(0, rpw, row_body, 0)

    zero16 = jnp.zeros((_L,), jnp.float32)

    def z_body(i, _):
        for j in range(d // _L):
            agg_v[i, pl.ds(j * _L, _L)] = zero16
        return 0

    lax.fori_loop(0, rpw, z_body, 0)

    k0 = s0 // _CH
    k1 = (s1 + _CH - 1) // _CH

    def chunk_body(k, _):
        b = k * _CH
        pltpu.async_copy(wx_hbm.at[sdep_v.at[pl.ds(b, _CH)]], gbuf, sem).wait()

        def e_body(e, _):
            p = b + e

            def acc(_):
                row = sload(sgov_v, p) - r0
                av = sload(val_v, p)
                for j in range(d // _L):
                    cur = agg_v[row, pl.ds(j * _L, _L)]
                    agg_v[row, pl.ds(j * _L, _L)] = (
                        cur + av * gbuf[e, pl.ds(j * _L, _L)])
                return 0

            lax.cond((p >= s0) & (p < s1), acc, lambda _: 0, 0)
            return 0

        lax.fori_loop(0, _CH, e_body, 0)
        return 0

    lax.fori_loop(k0, k1, chunk_body, 0)

    pltpu.async_copy(wx_hbm.at[h0gov_v], hbuf, sem).wait()

    def o_body(i, _):
        for j in range(d // _L):
            t = agg_v[i, pl.ds(j * _L, _L)] + hbuf[i, pl.ds(j * _L, _L)]
            hbuf[i, pl.ds(j * _L, _L)] = jnp.where(t >= 0, t, 0.2 * t)
        return 0

    lax.fori_loop(0, rpw, o_body, 0)
    pltpu.async_copy(hbuf, out_hbm.at[outpos_v], sem).wait()


def kernel(_input, dependency_triples, W, a):
    n, d = _input.shape
    m = dependency_triples.shape[0]
    rpw = n // _NW

    deps = dependency_triples[:, 0].astype(jnp.int32)
    govs = dependency_triples[:, 2].astype(jnp.int32)
    idx = jnp.arange(m, dtype=jnp.int32)

    kcell = govs * n + deps
    ord2 = jnp.argsort(kcell, stable=True)
    kc_s = kcell[ord2]
    sgov = govs[ord2]
    sdep = deps[ord2]
    is_last_s = jnp.concatenate(
        [kc_s[1:] != kc_s[:-1], jnp.ones((1,), bool)])
    t = jnp.where(is_last_s, idx, m)
    winner_pos = jnp.flip(lax.cummin(jnp.flip(t)))

    row_start = jnp.searchsorted(
        kc_s, jnp.arange(n, dtype=jnp.int32) * n).astype(jnp.int32)
    counts = (jnp.concatenate([row_start[1:], jnp.full((1,), m, jnp.int32)])
              - row_start)

    upd = jnp.stack([idx, m - 1 - idx], axis=1)
    ext = jnp.full((n, 2), -1, jnp.int32).at[deps].max(upd)
    last_idx = ext[:, 0]
    h0gov = govs[last_idx]
    first_idx = jnp.where(ext[:, 1] < 0, m, m - 1 - ext[:, 1])
    first_idx = first_idx.at[0].set(-1)
    keys = jnp.argsort(first_idx)
    outpos = jnp.zeros((n,), jnp.int32).at[keys].set(jnp.arange(n, dtype=jnp.int32))

    a2 = a.reshape(2, d)
    wx, sgd = pl.pallas_call(
        _tc_mm,
        out_shape=[jax.ShapeDtypeStruct((n, d), jnp.float32),
                   jax.ShapeDtypeStruct((n, 2), jnp.float32)],
    )(_input, W, a2)
    sg = sgd[:, 0]
    sd = sgd[:, 1]

    mesh = plsc.VectorSubcoreMesh(core_axis_name="c", subcore_axis_name="s")
    sc = pl.kernel(
        functools.partial(_sc_kernel, n, m, d, rpw),
        mesh=mesh,
        compiler_params=pltpu.CompilerParams(needs_layout_passes=False),
        out_type=jax.ShapeDtypeStruct((n, d), jnp.float32),
        scratch_types=[
            pltpu.VMEM((m + _EC,), jnp.int32),
            pltpu.VMEM((m + _EC,), jnp.int32),
            pltpu.VMEM((m + _EC,), jnp.int32),
            pltpu.VMEM((m + _L,), jnp.float32),
            pltpu.VMEM((m + _L,), jnp.float32),
            pltpu.VMEM((n,), jnp.float32),
            pltpu.VMEM((n,), jnp.float32),
            pltpu.VMEM((rpw + _L,), jnp.int32),
            pltpu.VMEM((rpw + _L,), jnp.int32),
            pltpu.VMEM((rpw,), jnp.int32),
            pltpu.VMEM((rpw,), jnp.int32),
            pltpu.VMEM((_CH, d), jnp.float32),
            pltpu.VMEM((rpw, d), jnp.float32),
            pltpu.VMEM((rpw, d), jnp.float32),
            pltpu.SemaphoreType.DMA,
        ],
    )
    pad = jnp.zeros((_EC,), jnp.int32)
    return sc(wx, sg, sd,
              jnp.concatenate([sgov, pad]),
              jnp.concatenate([sdep, pad]),
              jnp.concatenate([winner_pos, pad]),
              row_start, counts, h0gov, outpos)


# speedup vs baseline: 3.0446x; 1.0291x over previous
import functools

import jax
import jax.numpy as jnp
from jax import lax
from jax.experimental import pallas as pl
from jax.experimental.pallas import tpu as pltpu
from jax.experimental.pallas import tpu_sc as plsc

_L = 16
_NW = 32
_CH = 64
_EC = 512


def _tc_mm(x_ref, w_ref, a2_ref, wx_ref, sgd_ref):
    wx = lax.dot_general(x_ref[...], w_ref[...], (((1,), (1,)), ((), ())),
                         preferred_element_type=jnp.float32)
    wx_ref[...] = wx
    sgd_ref[...] = lax.dot_general(wx, a2_ref[...], (((1,), (1,)), ((), ())),
                                   preferred_element_type=jnp.float32)


def _sc_kernel(n, m, d, rpw,
               wx_hbm, sg_hbm, sd_hbm, sgov_hbm, sdep_hbm, winner_hbm,
               rs_hbm, cnt_hbm, h0gov_hbm, outpos_hbm, out_hbm,
               sgov_v, sdep_v, winner_v, score_v, val_v, sg_v, sd_v,
               rs_v, cnt_v, h0gov_v, outpos_v, gbuf, agg_v, hbuf,
               sem, sem_a, sem_b, sem_h):
    c_id = lax.axis_index("c")
    s_id = lax.axis_index("s")
    wid = s_id * 2 + c_id
    r0 = wid * rpw

    def sload(ref, i):
        return ref[pl.ds(i, _L)][0]

    pltpu.sync_copy(rs_hbm.at[pl.ds(r0, rpw)], rs_v.at[pl.ds(0, rpw)])
    pltpu.sync_copy(cnt_hbm.at[pl.ds(r0, rpw)], cnt_v.at[pl.ds(0, rpw)])
    pltpu.sync_copy(h0gov_hbm.at[pl.ds(r0, rpw)], h0gov_v)
    pltpu.sync_copy(outpos_hbm.at[pl.ds(r0, rpw)], outpos_v)
    pltpu.sync_copy(sg_hbm.at[pl.ds(r0, rpw)], sg_v.at[pl.ds(r0, rpw)])
    pltpu.sync_copy(sd_hbm, sd_v)

    s0 = sload(rs_v, 0)
    s1 = sload(rs_v, rpw - 1) + sload(cnt_v, rpw - 1)
    b0a = (s0 // _CH) * _CH
    nblk = (s1 - b0a + _L - 1) // _L

    def edma_body(k, _):
        b = b0a + k * _EC
        c1 = pltpu.async_copy(sgov_hbm.at[pl.ds(b, _EC)],
                              sgov_v.at[pl.ds(b, _EC)], sem)
        c2 = pltpu.async_copy(sdep_hbm.at[pl.ds(b, _EC)],
                              sdep_v.at[pl.ds(b, _EC)], sem_a)
        c3 = pltpu.async_copy(winner_hbm.at[pl.ds(b, _EC)],
                              winner_v.at[pl.ds(b, _EC)], sem_b)
        c1.wait()
        c2.wait()
        c3.wait()
        return 0

    lax.fori_loop(0, (s1 - b0a + _EC - 1) // _EC, edma_body, 0)

    h0_dma = pltpu.async_copy(wx_hbm.at[h0gov_v], hbuf, sem_h)

    iota = lax.iota(jnp.int32, _L)
    NEG = jnp.float32(-3.0e38)

    def score_body(i, _):
        b = b0a + i * _L
        gi = sgov_v[pl.ds(b, _L)]
        di = sdep_v[pl.ds(b, _L)]
        score_v[pl.ds(b, _L)] = (plsc.load_gather(sg_v, [gi]) +
                                 plsc.load_gather(sd_v, [di]))
        return 0

    lax.fori_loop(0, nblk, score_body, 0)

    def val_body(i, _):
        b = b0a + i * _L
        wv = winner_v[pl.ds(b, _L)]
        val_v[pl.ds(b, _L)] = plsc.load_gather(score_v, [wv])
        return 0

    lax.fori_loop(0, nblk, val_body, 0)

    def row_body(r, _):
        s = sload(rs_v, r)
        c = sload(cnt_v, r)
        e_end = s + c
        b0 = (s // _L) * _L
        nch = (e_end - b0 + _L - 1) // _L

        def max_body(k, mc):
            b = b0 + k * _L
            v = val_v[pl.ds(b, _L)]
            g = b + iota
            msk = (g >= s) & (g < e_end) & (v > 0)
            return jnp.maximum(mc, jnp.max(jnp.where(msk, v, NEG), axis=0))

        def soft_iter(it, mc):
            def sum_body(k, acc):
                b = b0 + k * _L
                v = val_v[pl.ds(b, _L)]
                w = winner_v[pl.ds(b, _L)]
                g = b + iota
                msk = (g >= s) & (g < e_end) & (v > 0) & (w == g)
                return acc + jnp.sum(jnp.where(msk, jnp.exp(v - mc), 0.0),
                                     axis=0)

            den = lax.fori_loop(0, nch, sum_body, jnp.float32(0.0))

            def upd_body(k, _):
                b = b0 + k * _L
                v = val_v[pl.ds(b, _L)]
                g = b + iota
                rm = (g >= s) & (g < e_end)
                pos = rm & (v > 0)
                val_v[pl.ds(b, _L)] = jnp.where(
                    pos, jnp.exp(v - mc) / den, jnp.where(rm, 0.0, v))
                return 0

            lax.fori_loop(0, nch, upd_body, 0)
            return (jnp.full((_L,), 1.0, jnp.float32) / den)[0]

        def do_rows(_):
            m0 = lax.fori_loop(0, nch, max_body, NEG)

            def do_soft(_):
                lax.fori_loop(0, c, soft_iter, m0)
                return 0

            def do_unif(_):
                u = jnp.float32(1.0 / n)

                def ub(k, _):
                    b = b0 + k * _L
                    v = val_v[pl.ds(b, _L)]
                    g = b + iota
                    rm = (g >= s) & (g < e_end)
                    val_v[pl.ds(b, _L)] = jnp.where(rm, u, v)
                    return 0

                lax.fori_loop(0, nch, ub, 0)
                return 0

            lax.cond(m0 > NEG, do_soft, do_unif, 0)
            return 0

        lax.cond(c > 0, do_rows, lambda _: 0, 0)
        return 0

    lax.fori_loop(0, rpw, row_body, 0)

    zero16 = jnp.zeros((_L,), jnp.float32)

    def z_body(i, _):
        for j in range(d // _L):
            agg_v[i, pl.ds(j * _L, _L)] = zero16
        return 0

    lax.fori_loop(0, rpw, z_body, 0)

    k0 = s0 // _CH
    k1 = (s1 + _CH - 1) // _CH

    def gdma(k, buf, s):
        return pltpu.async_copy(
            wx_hbm.at[sdep_v.at[pl.ds(k * _CH, _CH)]], buf, s)

    def issue(k, par):
        lax.cond(par == 0,
                 lambda _: (gdma(k, gbuf.at[0], sem_a), 0)[1],
                 lambda _: (gdma(k, gbuf.at[1], sem_b), 0)[1], 0)

    def acc_chunk(k, buf):
        b = k * _CH

        def e_body(e, _):
            p = b + e

            def acc(_):
                row = sload(sgov_v, p) - r0
                av = sload(val_v, p)
                for j in range(d // _L):
                    cur = agg_v[row, pl.ds(j * _L, _L)]
                    agg_v[row, pl.ds(j * _L, _L)] = (
                        cur + av * buf[e, pl.ds(j * _L, _L)])
                return 0

            lax.cond((p >= s0) & (p < s1), acc, lambda _: 0, 0)
            return 0

        lax.fori_loop(0, _CH, e_body, 0)

    lax.cond(k0 < k1, lambda _: issue(k0, 0) or 0, lambda _: 0, 0)

    def chunk_body(j, _):
        k = k0 + j
        par = j % 2
        lax.cond(k + 1 < k1, lambda _: issue(k + 1, 1 - par) or 0,
                 lambda _: 0, 0)

        def w0(_):
            pltpu.make_async_copy(
                wx_hbm.at[sdep_v.at[pl.ds(k * _CH, _CH)]],
                gbuf.at[0], sem_a).wait()
            acc_chunk(k, gbuf.at[0])
            return 0

        def w1(_):
            pltpu.make_async_copy(
                wx_hbm.at[sdep_v.at[pl.ds(k * _CH, _CH)]],
                gbuf.at[1], sem_b).wait()
            acc_chunk(k, gbuf.at[1])
            return 0

        lax.cond(par == 0, w0, w1, 0)
        return 0

    lax.fori_loop(0, k1 - k0, chunk_body, 0)

    h0_dma.wait()

    def o_body(i, _):
        for j in range(d // _L):
            t = agg_v[i, pl.ds(j * _L, _L)] + hbuf[i, pl.ds(j * _L, _L)]
            hbuf[i, pl.ds(j * _L, _L)] = jnp.where(t >= 0, t, 0.2 * t)
        return 0

    lax.fori_loop(0, rpw, o_body, 0)
    pltpu.async_copy(hbuf, out_hbm.at[outpos_v], sem).wait()


def kernel(_input, dependency_triples, W, a):
    n, d = _input.shape
    m = dependency_triples.shape[0]
    rpw = n // _NW

    deps = dependency_triples[:, 0].astype(jnp.int32)
    govs = dependency_triples[:, 2].astype(jnp.int32)
    idx = jnp.arange(m, dtype=jnp.int32)

    kcell = govs * n + deps
    ord2 = jnp.argsort(kcell, stable=True)
    kc_s = kcell[ord2]
    sgov = govs[ord2]
    sdep = deps[ord2]
    is_last_s = jnp.concatenate(
        [kc_s[1:] != kc_s[:-1], jnp.ones((1,), bool)])
    t = jnp.where(is_last_s, idx, m)
    winner_pos = jnp.flip(lax.cummin(jnp.flip(t)))

    row_start = jnp.searchsorted(
        kc_s, jnp.arange(n, dtype=jnp.int32) * n).astype(jnp.int32)
    counts = (jnp.concatenate([row_start[1:], jnp.full((1,), m, jnp.int32)])
              - row_start)

    upd = jnp.stack([idx, m - 1 - idx], axis=1)
    ext = jnp.full((n, 2), -1, jnp.int32).at[deps].max(upd)
    last_idx = ext[:, 0]
    h0gov = govs[last_idx]
    first_idx = jnp.where(ext[:, 1] < 0, m, m - 1 - ext[:, 1])
    first_idx = first_idx.at[0].set(-1)
    keys = jnp.argsort(first_idx)
    outpos = jnp.zeros((n,), jnp.int32).at[keys].set(jnp.arange(n, dtype=jnp.int32))

    a2 = a.reshape(2, d)
    wx, sgd = pl.pallas_call(
        _tc_mm,
        out_shape=[jax.ShapeDtypeStruct((n, d), jnp.float32),
                   jax.ShapeDtypeStruct((n, 2), jnp.float32)],
    )(_input, W, a2)
    sg = sgd[:, 0]
    sd = sgd[:, 1]

    mesh = plsc.VectorSubcoreMesh(core_axis_name="c", subcore_axis_name="s")
    sc = pl.kernel(
        functools.partial(_sc_kernel, n, m, d, rpw),
        mesh=mesh,
        compiler_params=pltpu.CompilerParams(needs_layout_passes=False),
        out_type=jax.ShapeDtypeStruct((n, d), jnp.float32),
        scratch_types=[
            pltpu.VMEM((m + _EC,), jnp.int32),
            pltpu.VMEM((m + _EC,), jnp.int32),
            pltpu.VMEM((m + _EC,), jnp.int32),
            pltpu.VMEM((m + _L,), jnp.float32),
            pltpu.VMEM((m + _L,), jnp.float32),
            pltpu.VMEM((n,), jnp.float32),
            pltpu.VMEM((n,), jnp.float32),
            pltpu.VMEM((rpw + _L,), jnp.int32),
            pltpu.VMEM((rpw + _L,), jnp.int32),
            pltpu.VMEM((rpw,), jnp.int32),
            pltpu.VMEM((rpw,), jnp.int32),
            pltpu.VMEM((2, _CH, d), jnp.float32),
            pltpu.VMEM((rpw, d), jnp.float32),
            pltpu.VMEM((rpw, d), jnp.float32),
            pltpu.SemaphoreType.DMA,
            pltpu.SemaphoreType.DMA,
            pltpu.SemaphoreType.DMA,
            pltpu.SemaphoreType.DMA,
        ],
    )
    pad = jnp.zeros((_EC,), jnp.int32)
    return sc(wx, sg, sd,
              jnp.concatenate([sgov, pad]),
              jnp.concatenate([sdep, pad]),
              jnp.concatenate([winner_pos, pad]),
              row_start, counts, h0gov, outpos)


# speedup vs baseline: 3.8115x; 1.2519x over previous
import functools

import jax
import jax.numpy as jnp
from jax import lax
from jax.experimental import pallas as pl
from jax.experimental.pallas import tpu as pltpu
from jax.experimental.pallas import tpu_sc as plsc

_L = 16
_NW = 32
_CH = 64
_EC = 512


def _tc_mm(x_ref, w_ref, a2_ref, wx_ref, sgd_ref):
    wx = lax.dot_general(x_ref[...], w_ref[...], (((1,), (1,)), ((), ())),
                         preferred_element_type=jnp.float32)
    wx_ref[...] = wx
    sgd_ref[...] = lax.dot_general(wx, a2_ref[...], (((1,), (1,)), ((), ())),
                                   preferred_element_type=jnp.float32)


def _occ_kernel(n, m, deps_hbm, govs_hbm, last_hbm, first_hbm,
                dchunk_v, gchunk_v, lastt_v, firstt_v, sem, sem_a):
    c_id = lax.axis_index("c")
    s_id = lax.axis_index("s")
    wid = s_id * 2 + c_id
    epw = m // _NW
    e0 = wid * epw

    c1 = pltpu.async_copy(deps_hbm.at[pl.ds(e0, epw)],
                          dchunk_v.at[pl.ds(0, epw)], sem)
    c2 = pltpu.async_copy(govs_hbm.at[pl.ds(e0, epw)],
                          gchunk_v.at[pl.ds(0, epw)], sem_a)

    neg1 = jnp.full((_L,), -1, jnp.int32)
    sentm = jnp.full((_L,), m, jnp.int32)

    def init_body(i, _):
        lastt_v[pl.ds(i * _L, _L)] = neg1
        firstt_v[pl.ds(i * _L, _L)] = sentm
        return 0

    lax.fori_loop(0, n // _L, init_body, 0)
    c1.wait()
    c2.wait()

    iota = lax.iota(jnp.int32, _L)

    def rmw(tbl, node, newval):
        base = (node // _L) * _L
        vec = tbl[pl.ds(base, _L)]
        tbl[pl.ds(base, _L)] = jnp.where(iota == node - base, newval, vec)

    def last_body(e, _):
        dn = dchunk_v[pl.ds(e, _L)][0]
        gv = gchunk_v[pl.ds(e, _L)][0]
        rmw(lastt_v, dn, (e0 + e) * n + gv)
        return 0

    lax.fori_loop(0, epw, last_body, 0)

    def first_body(i, _):
        e = epw - 1 - i
        dn = dchunk_v[pl.ds(e, _L)][0]
        rmw(firstt_v, dn, e0 + e)
        return 0

    lax.fori_loop(0, epw, first_body, 0)

    o1 = pltpu.async_copy(lastt_v.at[pl.ds(0, n)], last_hbm.at[wid], sem)
    o2 = pltpu.async_copy(firstt_v.at[pl.ds(0, n)], first_hbm.at[wid], sem_a)
    o1.wait()
    o2.wait()


def _sc_kernel(n, m, d, rpw,
               wx_hbm, sg_hbm, sd_hbm, sgov_hbm, sdep_hbm, winner_hbm,
               rs_hbm, cnt_hbm, h0gov_hbm, outpos_hbm, out_hbm,
               sgov_v, sdep_v, winner_v, score_v, val_v, sg_v, sd_v,
               rs_v, cnt_v, h0gov_v, outpos_v, gbuf, agg_v, hbuf,
               sem, sem_a, sem_b, sem_h):
    c_id = lax.axis_index("c")
    s_id = lax.axis_index("s")
    wid = s_id * 2 + c_id
    r0 = wid * rpw

    def sload(ref, i):
        return ref[pl.ds(i, _L)][0]

    pltpu.sync_copy(rs_hbm.at[pl.ds(r0, rpw)], rs_v.at[pl.ds(0, rpw)])
    pltpu.sync_copy(cnt_hbm.at[pl.ds(r0, rpw)], cnt_v.at[pl.ds(0, rpw)])
    pltpu.sync_copy(h0gov_hbm.at[pl.ds(r0, rpw)], h0gov_v)
    pltpu.sync_copy(outpos_hbm.at[pl.ds(r0, rpw)], outpos_v)
    pltpu.sync_copy(sg_hbm.at[pl.ds(r0, rpw)], sg_v.at[pl.ds(r0, rpw)])
    pltpu.sync_copy(sd_hbm, sd_v)

    s0 = sload(rs_v, 0)
    s1 = sload(rs_v, rpw - 1) + sload(cnt_v, rpw - 1)
    b0a = (s0 // _CH) * _CH
    nblk = (s1 - b0a + _L - 1) // _L

    def edma_body(k, _):
        b = b0a + k * _EC
        c1 = pltpu.async_copy(sgov_hbm.at[pl.ds(b, _EC)],
                              sgov_v.at[pl.ds(b, _EC)], sem)
        c2 = pltpu.async_copy(sdep_hbm.at[pl.ds(b, _EC)],
                              sdep_v.at[pl.ds(b, _EC)], sem_a)
        c3 = pltpu.async_copy(winner_hbm.at[pl.ds(b, _EC)],
                              winner_v.at[pl.ds(b, _EC)], sem_b)
        c1.wait()
        c2.wait()
        c3.wait()
        return 0

    lax.fori_loop(0, (s1 - b0a + _EC - 1) // _EC, edma_body, 0)

    h0_dma = pltpu.async_copy(wx_hbm.at[h0gov_v], hbuf, sem_h)

    iota = lax.iota(jnp.int32, _L)
    NEG = jnp.float32(-3.0e38)

    def score_body(i, _):
        b = b0a + i * _L
        gi = sgov_v[pl.ds(b, _L)]
        di = sdep_v[pl.ds(b, _L)]
        score_v[pl.ds(b, _L)] = (plsc.load_gather(sg_v, [gi]) +
                                 plsc.load_gather(sd_v, [di]))
        return 0

    lax.fori_loop(0, nblk, score_body, 0)

    def val_body(i, _):
        b = b0a + i * _L
        wv = winner_v[pl.ds(b, _L)]
        val_v[pl.ds(b, _L)] = plsc.load_gather(score_v, [wv])
        return 0

    lax.fori_loop(0, nblk, val_body, 0)

    def row_body(r, _):
        s = sload(rs_v, r)
        c = sload(cnt_v, r)
        e_end = s + c
        b0 = (s // _L) * _L
        nch = (e_end - b0 + _L - 1) // _L

        def max_body(k, mc):
            b = b0 + k * _L
            v = val_v[pl.ds(b, _L)]
            g = b + iota
            msk = (g >= s) & (g < e_end) & (v > 0)
            return jnp.maximum(mc, jnp.max(jnp.where(msk, v, NEG), axis=0))

        def soft_iter(it, mc):
            def sum_body(k, acc):
                b = b0 + k * _L
                v = val_v[pl.ds(b, _L)]
                w = winner_v[pl.ds(b, _L)]
                g = b + iota
                msk = (g >= s) & (g < e_end) & (v > 0) & (w == g)
                return acc + jnp.sum(jnp.where(msk, jnp.exp(v - mc), 0.0),
                                     axis=0)

            den = lax.fori_loop(0, nch, sum_body, jnp.float32(0.0))

            def upd_body(k, _):
                b = b0 + k * _L
                v = val_v[pl.ds(b, _L)]
                g = b + iota
                rm = (g >= s) & (g < e_end)
                pos = rm & (v > 0)
                val_v[pl.ds(b, _L)] = jnp.where(
                    pos, jnp.exp(v - mc) / den, jnp.where(rm, 0.0, v))
                return 0

            lax.fori_loop(0, nch, upd_body, 0)
            return (jnp.full((_L,), 1.0, jnp.float32) / den)[0]

        def do_rows(_):
            m0 = lax.fori_loop(0, nch, max_body, NEG)

            def do_soft(_):
                lax.fori_loop(0, c, soft_iter, m0)
                return 0

            def do_unif(_):
                u = jnp.float32(1.0 / n)

                def ub(k, _):
                    b = b0 + k * _L
                    v = val_v[pl.ds(b, _L)]
                    g = b + iota
                    rm = (g >= s) & (g < e_end)
                    val_v[pl.ds(b, _L)] = jnp.where(rm, u, v)
                    return 0

                lax.fori_loop(0, nch, ub, 0)
                return 0

            lax.cond(m0 > NEG, do_soft, do_unif, 0)
            return 0

        lax.cond(c > 0, do_rows, lambda _: 0, 0)
        return 0

    lax.fori_loop(0, rpw, row_body, 0)

    zero16 = jnp.zeros((_L,), jnp.float32)

    def z_body(i, _):
        for j in range(d // _L):
            agg_v[i, pl.ds(j * _L, _L)] = zero16
        return 0

    lax.fori_loop(0, rpw, z_body, 0)

    k0 = s0 // _CH
    k1 = (s1 + _CH - 1) // _CH

    def gdma(k, buf, s):
        return pltpu.async_copy(
            wx_hbm.at[sdep_v.at[pl.ds(k * _CH, _CH)]], buf, s)

    def issue(k, par):
        lax.cond(par == 0,
                 lambda _: (gdma(k, gbuf.at[0], sem_a), 0)[1],
                 lambda _: (gdma(k, gbuf.at[1], sem_b), 0)[1], 0)

    def acc_chunk(k, buf):
        b = k * _CH

        def e_body(e, _):
            p = b + e

            def acc(_):
                row = sload(sgov_v, p) - r0
                av = sload(val_v, p)
                for j in range(d // _L):
                    cur = agg_v[row, pl.ds(j * _L, _L)]
                    agg_v[row, pl.ds(j * _L, _L)] = (
                        cur + av * buf[e, pl.ds(j * _L, _L)])
                return 0

            lax.cond((p >= s0) & (p < s1), acc, lambda _: 0, 0)
            return 0

        lax.fori_loop(0, _CH, e_body, 0)

    lax.cond(k0 < k1, lambda _: issue(k0, 0) or 0, lambda _: 0, 0)

    def chunk_body(j, _):
        k = k0 + j
        par = j % 2
        lax.cond(k + 1 < k1, lambda _: issue(k + 1, 1 - par) or 0,
                 lambda _: 0, 0)

        def w0(_):
            pltpu.make_async_copy(
                wx_hbm.at[sdep_v.at[pl.ds(k * _CH, _CH)]],
                gbuf.at[0], sem_a).wait()
            acc_chunk(k, gbuf.at[0])
            return 0

        def w1(_):
            pltpu.make_async_copy(
                wx_hbm.at[sdep_v.at[pl.ds(k * _CH, _CH)]],
                gbuf.at[1], sem_b).wait()
            acc_chunk(k, gbuf.at[1])
            return 0

        lax.cond(par == 0, w0, w1, 0)
        return 0

    lax.fori_loop(0, k1 - k0, chunk_body, 0)

    h0_dma.wait()

    def o_body(i, _):
        for j in range(d // _L):
            t = agg_v[i, pl.ds(j * _L, _L)] + hbuf[i, pl.ds(j * _L, _L)]
            hbuf[i, pl.ds(j * _L, _L)] = jnp.where(t >= 0, t, 0.2 * t)
        return 0

    lax.fori_loop(0, rpw, o_body, 0)
    pltpu.async_copy(hbuf, out_hbm.at[outpos_v], sem).wait()


def kernel(_input, dependency_triples, W, a):
    n, d = _input.shape
    m = dependency_triples.shape[0]
    rpw = n // _NW

    deps = dependency_triples[:, 0].astype(jnp.int32)
    govs = dependency_triples[:, 2].astype(jnp.int32)
    idx = jnp.arange(m, dtype=jnp.int32)

    kcell = govs * n + deps
    ord2 = jnp.argsort(kcell, stable=True)
    kc_s = kcell[ord2]
    sgov = govs[ord2]
    sdep = deps[ord2]
    is_last_s = jnp.concatenate(
        [kc_s[1:] != kc_s[:-1], jnp.ones((1,), bool)])
    t = jnp.where(is_last_s, idx, m)
    winner_pos = jnp.flip(lax.cummin(jnp.flip(t)))

    row_start = jnp.searchsorted(
        kc_s, jnp.arange(n, dtype=jnp.int32) * n).astype(jnp.int32)
    counts = (jnp.concatenate([row_start[1:], jnp.full((1,), m, jnp.int32)])
              - row_start)

    mesh = plsc.VectorSubcoreMesh(core_axis_name="c", subcore_axis_name="s")
    occ = pl.kernel(
        functools.partial(_occ_kernel, n, m),
        mesh=mesh,
        compiler_params=pltpu.CompilerParams(needs_layout_passes=False),
        out_type=[jax.ShapeDtypeStruct((_NW, n), jnp.int32),
                  jax.ShapeDtypeStruct((_NW, n), jnp.int32)],
        scratch_types=[
            pltpu.VMEM((m // _NW + _L,), jnp.int32),
            pltpu.VMEM((m // _NW + _L,), jnp.int32),
            pltpu.VMEM((n,), jnp.int32),
            pltpu.VMEM((n,), jnp.int32),
            pltpu.SemaphoreType.DMA,
            pltpu.SemaphoreType.DMA,
        ],
    )
    last_tbl, first_tbl = occ(deps, govs)
    packed = jnp.max(last_tbl, axis=0)
    h0gov = packed % n
    first_idx = jnp.min(first_tbl, axis=0)
    first_idx = first_idx.at[0].set(-1)
    keys = jnp.argsort(first_idx)
    outpos = jnp.zeros((n,), jnp.int32).at[keys].set(jnp.arange(n, dtype=jnp.int32))

    a2 = a.reshape(2, d)
    wx, sgd = pl.pallas_call(
        _tc_mm,
        out_shape=[jax.ShapeDtypeStruct((n, d), jnp.float32),
                   jax.ShapeDtypeStruct((n, 2), jnp.float32)],
    )(_input, W, a2)
    sg = sgd[:, 0]
    sd = sgd[:, 1]

    sc = pl.kernel(
        functools.partial(_sc_kernel, n, m, d, rpw),
        mesh=mesh,
        compiler_params=pltpu.CompilerParams(needs_layout_passes=False),
        out_type=jax.ShapeDtypeStruct((n, d), jnp.float32),
        scratch_types=[
            pltpu.VMEM((m + _EC,), jnp.int32),
            pltpu.VMEM((m + _EC,), jnp.int32),
            pltpu.VMEM((m + _EC,), jnp.int32),
            pltpu.VMEM((m + _L,), jnp.float32),
            pltpu.VMEM((m + _L,), jnp.float32),
            pltpu.VMEM((n,), jnp.float32),
            pltpu.VMEM((n,), jnp.float32),
            pltpu.VMEM((rpw + _L,), jnp.int32),
            pltpu.VMEM((rpw + _L,), jnp.int32),
            pltpu.VMEM((rpw,), jnp.int32),
            pltpu.VMEM((rpw,), jnp.int32),
            pltpu.VMEM((2, _CH, d), jnp.float32),
            pltpu.VMEM((rpw, d), jnp.float32),
            pltpu.VMEM((rpw, d), jnp.float32),
            pltpu.SemaphoreType.DMA,
            pltpu.SemaphoreType.DMA,
            pltpu.SemaphoreType.DMA,
            pltpu.SemaphoreType.DMA,
        ],
    )
    pad = jnp.zeros((_EC,), jnp.int32)
    return sc(wx, sg, sd,
              jnp.concatenate([sgov, pad]),
              jnp.concatenate([sdep, pad]),
              jnp.concatenate([winner_pos, pad]),
              row_start, counts, h0gov, outpos)


# speedup vs baseline: 5.4534x; 1.4308x over previous
import functools

import jax
import jax.numpy as jnp
from jax import lax
from jax.experimental import pallas as pl
from jax.experimental.pallas import tpu as pltpu
from jax.experimental.pallas import tpu_sc as plsc

_L = 16
_NW = 32
_CH = 64
_EC = 512


def _tc_mm(x_ref, w_ref, a2_ref, wx_ref, sgd_ref):
    wx = lax.dot_general(x_ref[...], w_ref[...], (((1,), (1,)), ((), ())),
                         preferred_element_type=jnp.float32)
    wx_ref[...] = wx
    sgd_ref[...] = lax.dot_general(wx, a2_ref[...], (((1,), (1,)), ((), ())),
                                   preferred_element_type=jnp.float32)


def _occ_kernel(n, m, deps_hbm, govs_hbm, last_hbm, first_hbm, cnt_hbm,
                dchunk_v, gchunk_v, lastt_v, firstt_v, cntt_v,
                sem, sem_a, sem_b):
    c_id = lax.axis_index("c")
    s_id = lax.axis_index("s")
    wid = s_id * 2 + c_id
    epw = m // _NW
    e0 = wid * epw

    c1 = pltpu.async_copy(deps_hbm.at[pl.ds(e0, epw)],
                          dchunk_v.at[pl.ds(0, epw)], sem)
    c2 = pltpu.async_copy(govs_hbm.at[pl.ds(e0, epw)],
                          gchunk_v.at[pl.ds(0, epw)], sem_a)

    neg1 = jnp.full((_L,), -1, jnp.int32)
    sentm = jnp.full((_L,), m, jnp.int32)
    zero = jnp.zeros((_L,), jnp.int32)

    def init_body(i, _):
        lastt_v[pl.ds(i * _L, _L)] = neg1
        firstt_v[pl.ds(i * _L, _L)] = sentm
        cntt_v[pl.ds(i * _L, _L)] = zero
        return 0

    lax.fori_loop(0, n // _L, init_body, 0)
    c1.wait()
    c2.wait()

    iota = lax.iota(jnp.int32, _L)

    def last_body(e, _):
        dn = dchunk_v[pl.ds(e, _L)][0]
        gv = gchunk_v[pl.ds(e, _L)][0]
        db = (dn // _L) * _L
        dvec = lastt_v[pl.ds(db, _L)]
        lastt_v[pl.ds(db, _L)] = jnp.where(
            iota == dn - db, (e0 + e) * n + gv, dvec)
        gb = (gv // _L) * _L
        gvec = cntt_v[pl.ds(gb, _L)]
        cntt_v[pl.ds(gb, _L)] = jnp.where(iota == gv - gb, gvec + 1, gvec)
        return 0

    lax.fori_loop(0, epw, last_body, 0)

    def first_body(i, _):
        e = epw - 1 - i
        dn = dchunk_v[pl.ds(e, _L)][0]
        db = (dn // _L) * _L
        dvec = firstt_v[pl.ds(db, _L)]
        firstt_v[pl.ds(db, _L)] = jnp.where(iota == dn - db, e0 + e, dvec)
        return 0

    lax.fori_loop(0, epw, first_body, 0)

    o1 = pltpu.async_copy(lastt_v.at[pl.ds(0, n)], last_hbm.at[wid], sem)
    o2 = pltpu.async_copy(firstt_v.at[pl.ds(0, n)], first_hbm.at[wid], sem_a)
    o3 = pltpu.async_copy(cntt_v.at[pl.ds(0, n)], cnt_hbm.at[wid], sem_b)
    o1.wait()
    o2.wait()
    o3.wait()


def _scat_kernel(n, m, mp2, deps_hbm, govs_hbm, offs_hbm, skey_hbm,
                 dchunk_v, gchunk_v, offs_v, dst_v, keys_v, sem, sem_a):
    c_id = lax.axis_index("c")
    s_id = lax.axis_index("s")
    wid = s_id * 2 + c_id
    epw = m // _NW
    e0 = wid * epw

    c1 = pltpu.async_copy(deps_hbm.at[pl.ds(e0, epw)],
                          dchunk_v.at[pl.ds(0, epw)], sem)
    c2 = pltpu.async_copy(offs_hbm.at[wid], offs_v.at[pl.ds(0, n)], sem_a)
    pltpu.sync_copy(govs_hbm.at[pl.ds(e0, epw)], gchunk_v.at[pl.ds(0, epw)])
    c1.wait()
    c2.wait()

    iota = lax.iota(jnp.int32, _L)

    def e_body(e, _):
        dn = dchunk_v[pl.ds(e, _L)][0]
        gv = gchunk_v[pl.ds(e, _L)][0]
        o = offs_v[pl.ds(gv, _L)][0]
        gb = (gv // _L) * _L
        gvec = offs_v[pl.ds(gb, _L)]
        offs_v[pl.ds(gb, _L)] = jnp.where(iota == gv - gb, o + 1, gvec)
        eb = (e // _L) * _L
        kvec = keys_v[pl.ds(eb, _L)]
        keys_v[pl.ds(eb, _L)] = jnp.where(iota == e - eb, gv * mp2 + dn, kvec)
        dvec = dst_v[pl.ds(eb, _L)]
        dst_v[pl.ds(eb, _L)] = jnp.where(iota == e - eb, o, dvec)
        return 0

    lax.fori_loop(0, epw, e_body, 0)
    pltpu.async_copy(keys_v, skey_hbm.at[dst_v], sem).wait()


def _sc_kernel(n, m, d, rpw, mp2,
               wx_hbm, sg_hbm, sd_hbm, skey_hbm,
               rs_hbm, cnt_hbm, h0gov_hbm, outpos_hbm, out_hbm,
               skey_v, sgov_v, sdep_v, winner_v, score_v, wtab_v, sg_v, sd_v,
               rs_v, cnt_v, h0gov_v, outpos_v, gbuf, agg_v, hbuf,
               sem, sem_a, sem_b, sem_h):
    c_id = lax.axis_index("c")
    s_id = lax.axis_index("s")
    wid = s_id * 2 + c_id
    r0 = wid * rpw

    def sload(ref, i):
        return ref[pl.ds(i, _L)][0]

    pltpu.sync_copy(rs_hbm.at[pl.ds(r0, rpw)], rs_v.at[pl.ds(0, rpw)])
    pltpu.sync_copy(cnt_hbm.at[pl.ds(r0, rpw)], cnt_v.at[pl.ds(0, rpw)])
    pltpu.sync_copy(h0gov_hbm.at[pl.ds(r0, rpw)], h0gov_v)
    pltpu.sync_copy(outpos_hbm.at[pl.ds(r0, rpw)], outpos_v)
    pltpu.sync_copy(sg_hbm.at[pl.ds(r0, rpw)], sg_v.at[pl.ds(r0, rpw)])
    pltpu.sync_copy(sd_hbm, sd_v.at[pl.ds(0, n)])

    s0 = sload(rs_v, 0)
    s1 = sload(rs_v, rpw - 1) + sload(cnt_v, rpw - 1)
    b0a = (s0 // _CH) * _CH

    def edma_body(k, _):
        b = b0a + k * _EC
        pltpu.sync_copy(skey_hbm.at[pl.ds(b, _EC)], skey_v.at[pl.ds(b, _EC)])
        return 0

    lax.fori_loop(0, (s1 - b0a + _EC - 1) // _EC, edma_body, 0)

    h0_dma = pltpu.async_copy(wx_hbm.at[h0gov_v], hbuf, sem_h)

    iota = lax.iota(jnp.int32, _L)
    NEG = jnp.float32(-3.0e38)
    sh = mp2.bit_length() - 1

    nblk64 = ((s1 + _CH - 1) // _CH * _CH - b0a) // _L

    def score_body(i, _):
        b = b0a + i * _L
        kv = skey_v[pl.ds(b, _L)]
        gi = lax.shift_right_logical(kv, sh) & (mp2 - 1)
        di = kv & (mp2 - 1)
        sgov_v[pl.ds(b, _L)] = gi
        sdep_v[pl.ds(b, _L)] = di
        score_v[pl.ds(b, _L)] = (plsc.load_gather(sg_v, [gi]) +
                                 plsc.load_gather(sd_v, [di]))
        return 0

    lax.fori_loop(0, nblk64, score_body, 0)

    def row_body(r, _):
        s = sload(rs_v, r)
        c = sload(cnt_v, r)
        e_end = s + c
        b0 = (s // _L) * _L
        nch = (e_end - b0 + _L - 1) // _L

        def max_body(k, mc):
            b = b0 + k * _L
            v = score_v[pl.ds(b, _L)]
            g = b + iota
            msk = (g >= s) & (g < e_end) & (v > 0)
            return jnp.maximum(mc, jnp.max(jnp.where(msk, v, NEG), axis=0))

        def soft_iter(it, mc):
            def sum_body(k, acc):
                b = b0 + k * _L
                v = score_v[pl.ds(b, _L)]
                w = winner_v[pl.ds(b, _L)]
                g = b + iota
                msk = (g >= s) & (g < e_end) & (v > 0) & (w == g)
                return acc + jnp.sum(jnp.where(msk, jnp.exp(v - mc), 0.0),
                                     axis=0)

            den = lax.fori_loop(0, nch, sum_body, jnp.float32(0.0))

            def upd_body(k, _):
                b = b0 + k * _L
                v = score_v[pl.ds(b, _L)]
                g = b + iota
                rm = (g >= s) & (g < e_end)
                pos = rm & (v > 0)
                score_v[pl.ds(b, _L)] = jnp.where(
                    pos, jnp.exp(v - mc) / den, jnp.where(rm, 0.0, v))
                return 0

            lax.fori_loop(0, nch, upd_body, 0)
            return (jnp.full((_L,), 1.0, jnp.float32) / den)[0]

        def do_rows(_):
            m0 = lax.fori_loop(0, nch, max_body, NEG)

            def do_soft(_):
                def dict_body(p, _):
                    dn = sload(sdep_v, p)
                    db = (dn // _L) * _L
                    dvec = wtab_v[pl.ds(db, _L)]
                    wtab_v[pl.ds(db, _L)] = jnp.where(iota == dn - db, p, dvec)
                    return 0

                lax.fori_loop(s, e_end, dict_body, 0)

                def wg_body(k, _):
                    b = b0 + k * _L
                    di = sdep_v[pl.ds(b, _L)]
                    winner_v[pl.ds(b, _L)] = plsc.load_gather(wtab_v, [di])
                    return 0

                lax.fori_loop(0, nch, wg_body, 0)
                lax.fori_loop(0, c, soft_iter, m0)
                return 0

            def do_unif(_):
                u = jnp.float32(1.0 / n)

                def ub(k, _):
                    b = b0 + k * _L
                    v = score_v[pl.ds(b, _L)]
                    g = b + iota
                    rm = (g >= s) & (g < e_end)
                    score_v[pl.ds(b, _L)] = jnp.where(rm, u, v)
                    return 0

                lax.fori_loop(0, nch, ub, 0)
                return 0

            lax.cond(m0 > NEG, do_soft, do_unif, 0)
            return 0

        lax.cond(c > 0, do_rows, lambda _: 0, 0)
        return 0

    lax.fori_loop(0, rpw, row_body, 0)

    zero16 = jnp.zeros((_L,), jnp.float32)

    def z_body(i, _):
        for j in range(d // _L):
            agg_v[i, pl.ds(j * _L, _L)] = zero16
        return 0

    lax.fori_loop(0, rpw, z_body, 0)

    k0 = s0 // _CH
    k1 = (s1 + _CH - 1) // _CH

    def gdma(k, buf, s):
        return pltpu.async_copy(
            wx_hbm.at[sdep_v.at[pl.ds(k * _CH, _CH)]], buf, s)

    def issue(k, par):
        lax.cond(par == 0,
                 lambda _: (gdma(k, gbuf.at[0], sem_a), 0)[1],
                 lambda _: (gdma(k, gbuf.at[1], sem_b), 0)[1], 0)

    def acc_chunk(k, buf):
        b = k * _CH

        def e_body(e, _):
            p = b + e

            def acc(_):
                row = sload(sgov_v, p) - r0
                av = sload(score_v, p)
                for j in range(d // _L):
                    cur = agg_v[row, pl.ds(j * _L, _L)]
                    agg_v[row, pl.ds(j * _L, _L)] = (
                        cur + av * buf[e, pl.ds(j * _L, _L)])
                return 0

            lax.cond((p >= s0) & (p < s1), acc, lambda _: 0, 0)
            return 0

        lax.fori_loop(0, _CH, e_body, 0)

    lax.cond(k0 < k1, lambda _: issue(k0, 0) or 0, lambda _: 0, 0)

    def chunk_body(j, _):
        k = k0 + j
        par = j % 2
        lax.cond(k + 1 < k1, lambda _: issue(k + 1, 1 - par) or 0,
                 lambda _: 0, 0)

        def w0(_):
            pltpu.make_async_copy(
                wx_hbm.at[sdep_v.at[pl.ds(k * _CH, _CH)]],
                gbuf.at[0], sem_a).wait()
            acc_chunk(k, gbuf.at[0])
            return 0

        def w1(_):
            pltpu.make_async_copy(
                wx_hbm.at[sdep_v.at[pl.ds(k * _CH, _CH)]],
                gbuf.at[1], sem_b).wait()
            acc_chunk(k, gbuf.at[1])
            return 0

        lax.cond(par == 0, w0, w1, 0)
        return 0

    lax.fori_loop(0, k1 - k0, chunk_body, 0)

    h0_dma.wait()

    def o_body(i, _):
        for j in range(d // _L):
            t = agg_v[i, pl.ds(j * _L, _L)] + hbuf[i, pl.ds(j * _L, _L)]
            hbuf[i, pl.ds(j * _L, _L)] = jnp.where(t >= 0, t, 0.2 * t)
        return 0

    lax.fori_loop(0, rpw, o_body, 0)
    pltpu.async_copy(hbuf, out_hbm.at[outpos_v], sem).wait()


def kernel(_input, dependency_triples, W, a):
    n, d = _input.shape
    m = dependency_triples.shape[0]
    rpw = n // _NW

    deps = dependency_triples[:, 0].astype(jnp.int32)
    govs = dependency_triples[:, 2].astype(jnp.int32)

    mp2 = 1 << max(1, (n - 1).bit_length())
    mesh = plsc.VectorSubcoreMesh(core_axis_name="c", subcore_axis_name="s")
    occ = pl.kernel(
        functools.partial(_occ_kernel, n, m),
        mesh=mesh,
        compiler_params=pltpu.CompilerParams(needs_layout_passes=False),
        out_type=[jax.ShapeDtypeStruct((_NW, n), jnp.int32),
                  jax.ShapeDtypeStruct((_NW, n), jnp.int32),
                  jax.ShapeDtypeStruct((_NW, n), jnp.int32)],
        scratch_types=[
            pltpu.VMEM((m // _NW + _L,), jnp.int32),
            pltpu.VMEM((m // _NW + _L,), jnp.int32),
            pltpu.VMEM((n,), jnp.int32),
            pltpu.VMEM((n,), jnp.int32),
            pltpu.VMEM((n,), jnp.int32),
            pltpu.SemaphoreType.DMA,
            pltpu.SemaphoreType.DMA,
            pltpu.SemaphoreType.DMA,
        ],
    )
    last_tbl, first_tbl, cnt_tbl = occ(deps, govs)
    packed = jnp.max(last_tbl, axis=0)
    h0gov = packed % n
    first_idx = jnp.min(first_tbl, axis=0)
    first_idx = first_idx.at[0].set(-1)
    keys = jnp.argsort(first_idx)
    outpos = jnp.zeros((n,), jnp.int32).at[keys].set(jnp.arange(n, dtype=jnp.int32))

    counts = jnp.sum(cnt_tbl, axis=0)
    csum = jnp.cumsum(counts).astype(jnp.int32)
    row_start = csum - counts
    offs = (row_start[None, :]
            + jnp.cumsum(cnt_tbl, axis=0).astype(jnp.int32) - cnt_tbl)

    scat = pl.kernel(
        functools.partial(_scat_kernel, n, m, mp2),
        mesh=mesh,
        compiler_params=pltpu.CompilerParams(needs_layout_passes=False),
        out_type=jax.ShapeDtypeStruct((m + _EC,), jnp.int32),
        scratch_types=[
            pltpu.VMEM((m // _NW + _L,), jnp.int32),
            pltpu.VMEM((m // _NW + _L,), jnp.int32),
            pltpu.VMEM((n + _L,), jnp.int32),
            pltpu.VMEM((m // _NW,), jnp.int32),
            pltpu.VMEM((m // _NW,), jnp.int32),
            pltpu.SemaphoreType.DMA,
            pltpu.SemaphoreType.DMA,
        ],
    )
    skey = scat(deps, govs, offs)

    a2 = a.reshape(2, d)
    wx, sgd = pl.pallas_call(
        _tc_mm,
        out_shape=[jax.ShapeDtypeStruct((n, d), jnp.float32),
                   jax.ShapeDtypeStruct((n, 2), jnp.float32)],
    )(_input, W, a2)
    sg = sgd[:, 0]
    sd = sgd[:, 1]

    sc = pl.kernel(
        functools.partial(_sc_kernel, n, m, d, rpw, mp2),
        mesh=mesh,
        compiler_params=pltpu.CompilerParams(needs_layout_passes=False),
        out_type=jax.ShapeDtypeStruct((n, d), jnp.float32),
        scratch_types=[
            pltpu.VMEM((m + _EC,), jnp.int32),
            pltpu.VMEM((m + _L,), jnp.int32),
            pltpu.VMEM((m + _L,), jnp.int32),
            pltpu.VMEM((m + _L,), jnp.int32),
            pltpu.VMEM((m + _L,), jnp.float32),
            pltpu.VMEM((mp2,), jnp.int32),
            pltpu.VMEM((mp2,), jnp.float32),
            pltpu.VMEM((mp2,), jnp.float32),
            pltpu.VMEM((rpw + _L,), jnp.int32),
            pltpu.VMEM((rpw + _L,), jnp.int32),
            pltpu.VMEM((rpw,), jnp.int32),
            pltpu.VMEM((rpw,), jnp.int32),
            pltpu.VMEM((2, _CH, d), jnp.float32),
            pltpu.VMEM((rpw, d), jnp.float32),
            pltpu.VMEM((rpw, d), jnp.float32),
            pltpu.SemaphoreType.DMA,
            pltpu.SemaphoreType.DMA,
            pltpu.SemaphoreType.DMA,
            pltpu.SemaphoreType.DMA,
        ],
    )
    return sc(wx, sg, sd, skey, row_start, counts, h0gov, outpos)


# speedup vs baseline: 5.6458x; 1.0353x over previous
import functools

import jax
import jax.numpy as jnp
from jax import lax
from jax.experimental import pallas as pl
from jax.experimental.pallas import tpu as pltpu
from jax.experimental.pallas import tpu_sc as plsc

_L = 16
_NW = 32
_CH = 64
_EC = 512


def _tc_mm(x_ref, w_ref, a2_ref, wx_ref, sgd_ref):
    wx = lax.dot_general(x_ref[...], w_ref[...], (((1,), (1,)), ((), ())),
                         preferred_element_type=jnp.float32)
    wx_ref[...] = wx
    sgd_ref[...] = lax.dot_general(wx, a2_ref[...], (((1,), (1,)), ((), ())),
                                   preferred_element_type=jnp.float32)


def _occ_kernel(n, m, deps_hbm, govs_hbm, last_hbm, first_hbm, cnt_hbm,
                rank_hbm, dchunk_v, gchunk_v, lastt_v, firstt_v, cntt_v,
                rankt_v, sem, sem_a, sem_b):
    c_id = lax.axis_index("c")
    s_id = lax.axis_index("s")
    wid = s_id * 2 + c_id
    epw = m // _NW
    e0 = wid * epw

    c1 = pltpu.async_copy(deps_hbm.at[pl.ds(e0, epw)],
                          dchunk_v.at[pl.ds(0, epw)], sem)
    c2 = pltpu.async_copy(govs_hbm.at[pl.ds(e0, epw)],
                          gchunk_v.at[pl.ds(0, epw)], sem_a)

    neg1 = jnp.full((_L,), -1, jnp.int32)
    sentm = jnp.full((_L,), m, jnp.int32)
    zero = jnp.zeros((_L,), jnp.int32)

    def init_body(i, _):
        lastt_v[pl.ds(i * _L, _L)] = neg1
        firstt_v[pl.ds(i * _L, _L)] = sentm
        cntt_v[pl.ds(i * _L, _L)] = zero
        return 0

    lax.fori_loop(0, n // _L, init_body, 0)
    c1.wait()
    c2.wait()

    iota = lax.iota(jnp.int32, _L)

    def last_body(e, _):
        dn = dchunk_v[pl.ds(e, _L)][0]
        gv = gchunk_v[pl.ds(e, _L)][0]
        db = (dn // _L) * _L
        dvec = lastt_v[pl.ds(db, _L)]
        lastt_v[pl.ds(db, _L)] = jnp.where(
            iota == dn - db, (e0 + e) * n + gv, dvec)
        gb = (gv // _L) * _L
        gvec = cntt_v[pl.ds(gb, _L)]
        rk = jnp.max(jnp.where(iota == gv - gb, gvec, 0), axis=0)
        cntt_v[pl.ds(gb, _L)] = jnp.where(iota == gv - gb, gvec + 1, gvec)
        eb = (e // _L) * _L
        rvec = rankt_v[pl.ds(eb, _L)]
        rankt_v[pl.ds(eb, _L)] = jnp.where(iota == e - eb, rk, rvec)
        return 0

    lax.fori_loop(0, epw, last_body, 0)

    def first_body(i, _):
        e = epw - 1 - i
        dn = dchunk_v[pl.ds(e, _L)][0]
        db = (dn // _L) * _L
        dvec = firstt_v[pl.ds(db, _L)]
        firstt_v[pl.ds(db, _L)] = jnp.where(iota == dn - db, e0 + e, dvec)
        return 0

    lax.fori_loop(0, epw, first_body, 0)

    o1 = pltpu.async_copy(lastt_v.at[pl.ds(0, n)], last_hbm.at[wid], sem)
    o2 = pltpu.async_copy(firstt_v.at[pl.ds(0, n)], first_hbm.at[wid], sem_a)
    o3 = pltpu.async_copy(cntt_v.at[pl.ds(0, n)], cnt_hbm.at[wid], sem_b)
    pltpu.sync_copy(rankt_v.at[pl.ds(0, epw)], rank_hbm.at[pl.ds(e0, epw)])
    o1.wait()
    o2.wait()
    o3.wait()


def _scat_kernel(n, m, mp2, deps_hbm, govs_hbm, offs_hbm, rank_hbm, skey_hbm,
                 dchunk_v, gchunk_v, rchunk_v, offs_v, dst_v, keys_v,
                 sem, sem_a, sem_b):
    c_id = lax.axis_index("c")
    s_id = lax.axis_index("s")
    wid = s_id * 2 + c_id
    epw = m // _NW
    e0 = wid * epw

    c1 = pltpu.async_copy(deps_hbm.at[pl.ds(e0, epw)],
                          dchunk_v.at[pl.ds(0, epw)], sem)
    c2 = pltpu.async_copy(offs_hbm.at[wid], offs_v.at[pl.ds(0, n)], sem_a)
    c3 = pltpu.async_copy(rank_hbm.at[pl.ds(e0, epw)],
                          rchunk_v.at[pl.ds(0, epw)], sem_b)
    pltpu.sync_copy(govs_hbm.at[pl.ds(e0, epw)], gchunk_v.at[pl.ds(0, epw)])
    c1.wait()
    c2.wait()
    c3.wait()

    def blk_body(i, _):
        b = i * _L
        gvec = gchunk_v[pl.ds(b, _L)]
        dvec = dchunk_v[pl.ds(b, _L)]
        rvec = rchunk_v[pl.ds(b, _L)]
        keys_v[pl.ds(b, _L)] = gvec * mp2 + dvec
        dst_v[pl.ds(b, _L)] = plsc.load_gather(offs_v, [gvec]) + rvec
        return 0

    lax.fori_loop(0, epw // _L, blk_body, 0)
    pltpu.async_copy(keys_v, skey_hbm.at[dst_v], sem).wait()


def _sc_kernel(n, m, d, rpw, mp2,
               wx_hbm, sg_hbm, sd_hbm, skey_hbm,
               rs_hbm, cnt_hbm, h0gov_hbm, outpos_hbm, out_hbm,
               skey_v, sgov_v, sdep_v, winner_v, score_v, wtab_v, sg_v, sd_v,
               rs_v, cnt_v, h0gov_v, outpos_v, gbuf, agg_v, hbuf,
               sem, sem_a, sem_b, sem_h):
    c_id = lax.axis_index("c")
    s_id = lax.axis_index("s")
    wid = s_id * 2 + c_id
    r0 = wid * rpw

    def sload(ref, i):
        return ref[pl.ds(i, _L)][0]

    pltpu.sync_copy(rs_hbm.at[pl.ds(r0, rpw)], rs_v.at[pl.ds(0, rpw)])
    pltpu.sync_copy(cnt_hbm.at[pl.ds(r0, rpw)], cnt_v.at[pl.ds(0, rpw)])
    pltpu.sync_copy(h0gov_hbm.at[pl.ds(r0, rpw)], h0gov_v)
    pltpu.sync_copy(outpos_hbm.at[pl.ds(r0, rpw)], outpos_v)
    pltpu.sync_copy(sg_hbm.at[pl.ds(r0, rpw)], sg_v.at[pl.ds(r0, rpw)])
    pltpu.sync_copy(sd_hbm, sd_v.at[pl.ds(0, n)])

    s0 = sload(rs_v, 0)
    s1 = sload(rs_v, rpw - 1) + sload(cnt_v, rpw - 1)
    b0a = (s0 // _CH) * _CH

    def edma_body(k, _):
        b = b0a + k * _EC
        pltpu.sync_copy(skey_hbm.at[pl.ds(b, _EC)], skey_v.at[pl.ds(b, _EC)])
        return 0

    lax.fori_loop(0, (s1 - b0a + _EC - 1) // _EC, edma_body, 0)

    h0_dma = pltpu.async_copy(wx_hbm.at[h0gov_v], hbuf, sem_h)

    iota = lax.iota(jnp.int32, _L)
    NEG = jnp.float32(-3.0e38)
    sh = mp2.bit_length() - 1

    nblk64 = ((s1 + _CH - 1) // _CH * _CH - b0a) // _L

    def score_body(i, _):
        b = b0a + i * _L
        kv = skey_v[pl.ds(b, _L)]
        gi = lax.shift_right_logical(kv, sh) & (mp2 - 1)
        di = kv & (mp2 - 1)
        sgov_v[pl.ds(b, _L)] = gi
        sdep_v[pl.ds(b, _L)] = di
        score_v[pl.ds(b, _L)] = (plsc.load_gather(sg_v, [gi]) +
                                 plsc.load_gather(sd_v, [di]))
        return 0

    lax.fori_loop(0, nblk64, score_body, 0)

    def row_body(r, _):
        s = sload(rs_v, r)
        c = sload(cnt_v, r)
        e_end = s + c
        b0 = (s // _L) * _L
        nch = (e_end - b0 + _L - 1) // _L

        def max_body(k, mc):
            b = b0 + k * _L
            v = score_v[pl.ds(b, _L)]
            g = b + iota
            msk = (g >= s) & (g < e_end) & (v > 0)
            return jnp.maximum(mc, jnp.max(jnp.where(msk, v, NEG), axis=0))

        def soft_iter(it, mc):
            def sum_body(k, acc):
                b = b0 + k * _L
                v = score_v[pl.ds(b, _L)]
                w = winner_v[pl.ds(b, _L)]
                g = b + iota
                msk = (g >= s) & (g < e_end) & (v > 0) & (w == g)
                return acc + jnp.sum(jnp.where(msk, jnp.exp(v - mc), 0.0),
                                     axis=0)

            den = lax.fori_loop(0, nch, sum_body, jnp.float32(0.0))

            def upd_body(k, _):
                b = b0 + k * _L
                v = score_v[pl.ds(b, _L)]
                g = b + iota
                rm = (g >= s) & (g < e_end)
                pos = rm & (v > 0)
                score_v[pl.ds(b, _L)] = jnp.where(
                    pos, jnp.exp(v - mc) / den, jnp.where(rm, 0.0, v))
                return 0

            lax.fori_loop(0, nch, upd_body, 0)
            return (jnp.full((_L,), 1.0, jnp.float32) / den)[0]

        def do_rows(_):
            m0 = lax.fori_loop(0, nch, max_body, NEG)

            def do_soft(_):
                def dict_body(p, _):
                    dn = sload(sdep_v, p)
                    db = (dn // _L) * _L
                    dvec = wtab_v[pl.ds(db, _L)]
                    wtab_v[pl.ds(db, _L)] = jnp.where(iota == dn - db, p, dvec)
                    return 0

                lax.fori_loop(s, e_end, dict_body, 0)

                def wg_body(k, _):
                    b = b0 + k * _L
                    di = sdep_v[pl.ds(b, _L)]
                    winner_v[pl.ds(b, _L)] = plsc.load_gather(wtab_v, [di])
                    return 0

                lax.fori_loop(0, nch, wg_body, 0)
                lax.fori_loop(0, c, soft_iter, m0)
                return 0

            def do_unif(_):
                u = jnp.float32(1.0 / n)

                def ub(k, _):
                    b = b0 + k * _L
                    v = score_v[pl.ds(b, _L)]
                    g = b + iota
                    rm = (g >= s) & (g < e_end)
                    score_v[pl.ds(b, _L)] = jnp.where(rm, u, v)
                    return 0

                lax.fori_loop(0, nch, ub, 0)
                return 0

            lax.cond(m0 > NEG, do_soft, do_unif, 0)
            return 0

        lax.cond(c > 0, do_rows, lambda _: 0, 0)
        return 0

    lax.fori_loop(0, rpw, row_body, 0)

    zero16 = jnp.zeros((_L,), jnp.float32)

    def z_body(i, _):
        for j in range(d // _L):
            agg_v[i, pl.ds(j * _L, _L)] = zero16
        return 0

    lax.fori_loop(0, rpw, z_body, 0)

    k0 = s0 // _CH
    k1 = (s1 + _CH - 1) // _CH

    def gdma(k, buf, s):
        return pltpu.async_copy(
            wx_hbm.at[sdep_v.at[pl.ds(k * _CH, _CH)]], buf, s)

    def issue(k, par):
        lax.cond(par == 0,
                 lambda _: (gdma(k, gbuf.at[0], sem_a), 0)[1],
                 lambda _: (gdma(k, gbuf.at[1], sem_b), 0)[1], 0)

    def acc_chunk(k, buf):
        b = k * _CH

        def e_body(e, _):
            p = b + e

            def acc(_):
                row = sload(sgov_v, p) - r0
                av = sload(score_v, p)
                for j in range(d // _L):
                    cur = agg_v[row, pl.ds(j * _L, _L)]
                    agg_v[row, pl.ds(j * _L, _L)] = (
                        cur + av * buf[e, pl.ds(j * _L, _L)])
                return 0

            lax.cond((p >= s0) & (p < s1), acc, lambda _: 0, 0)
            return 0

        lax.fori_loop(0, _CH, e_body, 0)

    lax.cond(k0 < k1, lambda _: issue(k0, 0) or 0, lambda _: 0, 0)

    def chunk_body(j, _):
        k = k0 + j
        par = j % 2
        lax.cond(k + 1 < k1, lambda _: issue(k + 1, 1 - par) or 0,
                 lambda _: 0, 0)

        def w0(_):
            pltpu.make_async_copy(
                wx_hbm.at[sdep_v.at[pl.ds(k * _CH, _CH)]],
                gbuf.at[0], sem_a).wait()
            acc_chunk(k, gbuf.at[0])
            return 0

        def w1(_):
            pltpu.make_async_copy(
                wx_hbm.at[sdep_v.at[pl.ds(k * _CH, _CH)]],
                gbuf.at[1], sem_b).wait()
            acc_chunk(k, gbuf.at[1])
            return 0

        lax.cond(par == 0, w0, w1, 0)
        return 0

    lax.fori_loop(0, k1 - k0, chunk_body, 0)

    h0_dma.wait()

    def o_body(i, _):
        for j in range(d // _L):
            t = agg_v[i, pl.ds(j * _L, _L)] + hbuf[i, pl.ds(j * _L, _L)]
            hbuf[i, pl.ds(j * _L, _L)] = jnp.where(t >= 0, t, 0.2 * t)
        return 0

    lax.fori_loop(0, rpw, o_body, 0)
    pltpu.async_copy(hbuf, out_hbm.at[outpos_v], sem).wait()


def kernel(_input, dependency_triples, W, a):
    n, d = _input.shape
    m = dependency_triples.shape[0]
    rpw = n // _NW

    deps = dependency_triples[:, 0].astype(jnp.int32)
    govs = dependency_triples[:, 2].astype(jnp.int32)

    mp2 = 1 << max(1, (n - 1).bit_length())
    mesh = plsc.VectorSubcoreMesh(core_axis_name="c", subcore_axis_name="s")
    occ = pl.kernel(
        functools.partial(_occ_kernel, n, m),
        mesh=mesh,
        compiler_params=pltpu.CompilerParams(needs_layout_passes=False),
        out_type=[jax.ShapeDtypeStruct((_NW, n), jnp.int32),
                  jax.ShapeDtypeStruct((_NW, n), jnp.int32),
                  jax.ShapeDtypeStruct((_NW, n), jnp.int32),
                  jax.ShapeDtypeStruct((m,), jnp.int32)],
        scratch_types=[
            pltpu.VMEM((m // _NW + _L,), jnp.int32),
            pltpu.VMEM((m // _NW + _L,), jnp.int32),
            pltpu.VMEM((n,), jnp.int32),
            pltpu.VMEM((n,), jnp.int32),
            pltpu.VMEM((n,), jnp.int32),
            pltpu.VMEM((m // _NW,), jnp.int32),
            pltpu.SemaphoreType.DMA,
            pltpu.SemaphoreType.DMA,
            pltpu.SemaphoreType.DMA,
        ],
    )
    last_tbl, first_tbl, cnt_tbl, rank = occ(deps, govs)
    packed = jnp.max(last_tbl, axis=0)
    h0gov = packed % n
    first_idx = jnp.min(first_tbl, axis=0)
    first_idx = first_idx.at[0].set(-1)
    keys = jnp.argsort(first_idx)
    outpos = jnp.zeros((n,), jnp.int32).at[keys].set(jnp.arange(n, dtype=jnp.int32))

    counts = jnp.sum(cnt_tbl, axis=0)
    csum = jnp.cumsum(counts).astype(jnp.int32)
    row_start = csum - counts
    offs = (row_start[None, :]
            + jnp.cumsum(cnt_tbl, axis=0).astype(jnp.int32) - cnt_tbl)

    scat = pl.kernel(
        functools.partial(_scat_kernel, n, m, mp2),
        mesh=mesh,
        compiler_params=pltpu.CompilerParams(needs_layout_passes=False),
        out_type=jax.ShapeDtypeStruct((m + _EC,), jnp.int32),
        scratch_types=[
            pltpu.VMEM((m // _NW + _L,), jnp.int32),
            pltpu.VMEM((m // _NW + _L,), jnp.int32),
            pltpu.VMEM((m // _NW + _L,), jnp.int32),
            pltpu.VMEM((n + _L,), jnp.int32),
            pltpu.VMEM((m // _NW,), jnp.int32),
            pltpu.VMEM((m // _NW,), jnp.int32),
            pltpu.SemaphoreType.DMA,
            pltpu.SemaphoreType.DMA,
            pltpu.SemaphoreType.DMA,
        ],
    )
    skey = scat(deps, govs, offs, rank)

    a2 = a.reshape(2, d)
    wx, sgd = pl.pallas_call(
        _tc_mm,
        out_shape=[jax.ShapeDtypeStruct((n, d), jnp.float32),
                   jax.ShapeDtypeStruct((n, 2), jnp.float32)],
    )(_input, W, a2)
    sg = sgd[:, 0]
    sd = sgd[:, 1]

    sc = pl.kernel(
        functools.partial(_sc_kernel, n, m, d, rpw, mp2),
        mesh=mesh,
        compiler_params=pltpu.CompilerParams(needs_layout_passes=False),
        out_type=jax.ShapeDtypeStruct((n, d), jnp.float32),
        scratch_types=[
            pltpu.VMEM((m + _EC,), jnp.int32),
            pltpu.VMEM((m + _L,), jnp.int32),
            pltpu.VMEM((m + _L,), jnp.int32),
            pltpu.VMEM((m + _L,), jnp.int32),
            pltpu.VMEM((m + _L,), jnp.float32),
            pltpu.VMEM((mp2,), jnp.int32),
            pltpu.VMEM((mp2,), jnp.float32),
            pltpu.VMEM((mp2,), jnp.float32),
            pltpu.VMEM((rpw + _L,), jnp.int32),
            pltpu.VMEM((rpw + _L,), jnp.int32),
            pltpu.VMEM((rpw,), jnp.int32),
            pltpu.VMEM((rpw,), jnp.int32),
            pltpu.VMEM((2, _CH, d), jnp.float32),
            pltpu.VMEM((rpw, d), jnp.float32),
            pltpu.VMEM((rpw, d), jnp.float32),
            pltpu.SemaphoreType.DMA,
            pltpu.SemaphoreType.DMA,
            pltpu.SemaphoreType.DMA,
            pltpu.SemaphoreType.DMA,
        ],
    )
    return sc(wx, sg, sd, skey, row_start, counts, h0gov, outpos)


# speedup vs baseline: 5.9262x; 1.0497x over previous
import functools

import jax
import jax.numpy as jnp
from jax import lax
from jax.experimental import pallas as pl
from jax.experimental.pallas import tpu as pltpu
from jax.experimental.pallas import tpu_sc as plsc

_L = 16
_NW = 32
_CH = 64
_EC = 512


def _tc_mm(x_ref, w_ref, a2_ref, wx_ref, sgd_ref):
    wx = lax.dot_general(x_ref[...], w_ref[...], (((1,), (1,)), ((), ())),
                         preferred_element_type=jnp.float32)
    wx_ref[...] = wx
    sgd_ref[...] = lax.dot_general(wx, a2_ref[...], (((1,), (1,)), ((), ())),
                                   preferred_element_type=jnp.float32)


def _occ_kernel(n, m, deps_hbm, govs_hbm, last_hbm, first_hbm, cnt_hbm,
                rank_hbm, dchunk_v, gchunk_v, lastt_v, firstt_v, cntt_v,
                rankt_v, sem, sem_a, sem_b):
    c_id = lax.axis_index("c")
    s_id = lax.axis_index("s")
    wid = s_id * 2 + c_id
    epw = m // _NW
    e0 = wid * epw

    c1 = pltpu.async_copy(deps_hbm.at[pl.ds(e0, epw)],
                          dchunk_v.at[pl.ds(0, epw)], sem)
    c2 = pltpu.async_copy(govs_hbm.at[pl.ds(e0, epw)],
                          gchunk_v.at[pl.ds(0, epw)], sem_a)

    neg1 = jnp.full((_L,), -1, jnp.int32)
    sentm = jnp.full((_L,), m, jnp.int32)
    zero = jnp.zeros((_L,), jnp.int32)

    def init_body(i, _):
        lastt_v[pl.ds(i * _L, _L)] = neg1
        firstt_v[pl.ds(i * _L, _L)] = sentm
        cntt_v[pl.ds(i * _L, _L)] = zero
        return 0

    lax.fori_loop(0, n // _L, init_body, 0)
    c1.wait()
    c2.wait()

    iota = lax.iota(jnp.int32, _L)

    def edge_body(e, _):
        dn = dchunk_v[pl.ds(e, _L)][0]
        gv = gchunk_v[pl.ds(e, _L)][0]
        db = (dn // _L) * _L
        dmsk = iota == dn - db
        dvec = lastt_v[pl.ds(db, _L)]
        lastt_v[pl.ds(db, _L)] = jnp.where(dmsk, (e0 + e) * n + gv, dvec)
        fvec = firstt_v[pl.ds(db, _L)]
        firstt_v[pl.ds(db, _L)] = jnp.where(
            dmsk, jnp.minimum(fvec, e0 + e), fvec)
        gb = (gv // _L) * _L
        gvec = cntt_v[pl.ds(gb, _L)]
        rk = jnp.max(jnp.where(iota == gv - gb, gvec, 0), axis=0)
        cntt_v[pl.ds(gb, _L)] = jnp.where(iota == gv - gb, gvec + 1, gvec)
        eb = (e // _L) * _L
        rvec = rankt_v[pl.ds(eb, _L)]
        rankt_v[pl.ds(eb, _L)] = jnp.where(iota == e - eb, rk, rvec)
        return 0

    lax.fori_loop(0, epw, edge_body, 0)

    o1 = pltpu.async_copy(lastt_v.at[pl.ds(0, n)], last_hbm.at[wid], sem)
    o2 = pltpu.async_copy(firstt_v.at[pl.ds(0, n)], first_hbm.at[wid], sem_a)
    o3 = pltpu.async_copy(cntt_v.at[pl.ds(0, n)], cnt_hbm.at[wid], sem_b)
    pltpu.sync_copy(rankt_v.at[pl.ds(0, epw)], rank_hbm.at[pl.ds(e0, epw)])
    o1.wait()
    o2.wait()
    o3.wait()


def _scat_kernel(n, m, mp2, deps_hbm, govs_hbm, offs_hbm, rank_hbm, skey_hbm,
                 dchunk_v, gchunk_v, rchunk_v, offs_v, dst_v, keys_v,
                 sem, sem_a, sem_b):
    c_id = lax.axis_index("c")
    s_id = lax.axis_index("s")
    wid = s_id * 2 + c_id
    epw = m // _NW
    e0 = wid * epw

    c1 = pltpu.async_copy(deps_hbm.at[pl.ds(e0, epw)],
                          dchunk_v.at[pl.ds(0, epw)], sem)
    c2 = pltpu.async_copy(offs_hbm.at[wid], offs_v.at[pl.ds(0, n)], sem_a)
    c3 = pltpu.async_copy(rank_hbm.at[pl.ds(e0, epw)],
                          rchunk_v.at[pl.ds(0, epw)], sem_b)
    pltpu.sync_copy(govs_hbm.at[pl.ds(e0, epw)], gchunk_v.at[pl.ds(0, epw)])
    c1.wait()
    c2.wait()
    c3.wait()

    def blk_body(i, _):
        b = i * _L
        gvec = gchunk_v[pl.ds(b, _L)]
        dvec = dchunk_v[pl.ds(b, _L)]
        rvec = rchunk_v[pl.ds(b, _L)]
        keys_v[pl.ds(b, _L)] = gvec * mp2 + dvec
        dst_v[pl.ds(b, _L)] = plsc.load_gather(offs_v, [gvec]) + rvec
        return 0

    lax.fori_loop(0, epw // _L, blk_body, 0)
    pltpu.async_copy(keys_v, skey_hbm.at[dst_v], sem).wait()


def _sc_kernel(n, m, d, rpw, mp2,
               wx_hbm, sg_hbm, sd_hbm, skey_hbm,
               rs_hbm, cnt_hbm, h0gov_hbm, outpos_hbm, out_hbm,
               skey_v, sgov_v, sdep_v, winner_v, score_v, wtab_v, sg_v, sd_v,
               rs_v, cnt_v, h0gov_v, outpos_v, gbuf, agg_v, hbuf,
               sem, sem_a, sem_b, sem_h):
    c_id = lax.axis_index("c")
    s_id = lax.axis_index("s")
    wid = s_id * 2 + c_id
    r0 = wid * rpw

    def sload(ref, i):
        return ref[pl.ds(i, _L)][0]

    pltpu.sync_copy(rs_hbm.at[pl.ds(r0, rpw)], rs_v.at[pl.ds(0, rpw)])
    pltpu.sync_copy(cnt_hbm.at[pl.ds(r0, rpw)], cnt_v.at[pl.ds(0, rpw)])
    pltpu.sync_copy(h0gov_hbm.at[pl.ds(r0, rpw)], h0gov_v)
    pltpu.sync_copy(outpos_hbm.at[pl.ds(r0, rpw)], outpos_v)
    pltpu.sync_copy(sg_hbm.at[pl.ds(r0, rpw)], sg_v.at[pl.ds(r0, rpw)])
    pltpu.sync_copy(sd_hbm, sd_v.at[pl.ds(0, n)])

    s0 = sload(rs_v, 0)
    s1 = sload(rs_v, rpw - 1) + sload(cnt_v, rpw - 1)
    b0a = (s0 // _CH) * _CH

    def edma_body(k, _):
        b = b0a + k * _EC
        pltpu.sync_copy(skey_hbm.at[pl.ds(b, _EC)], skey_v.at[pl.ds(b, _EC)])
        return 0

    lax.fori_loop(0, (s1 - b0a + _EC - 1) // _EC, edma_body, 0)

    h0_dma = pltpu.async_copy(wx_hbm.at[h0gov_v], hbuf, sem_h)

    iota = lax.iota(jnp.int32, _L)
    NEG = jnp.float32(-3.0e38)
    sh = mp2.bit_length() - 1

    nblk64 = ((s1 + _CH - 1) // _CH * _CH - b0a) // _L

    def score_body(i, _):
        b = b0a + i * _L
        kv = skey_v[pl.ds(b, _L)]
        gi = lax.shift_right_logical(kv, sh) & (mp2 - 1)
        di = kv & (mp2 - 1)
        sgov_v[pl.ds(b, _L)] = gi
        sdep_v[pl.ds(b, _L)] = di
        score_v[pl.ds(b, _L)] = (plsc.load_gather(sg_v, [gi]) +
                                 plsc.load_gather(sd_v, [di]))
        return 0

    lax.fori_loop(0, nblk64, score_body, 0)

    def row_body(r, _):
        s = sload(rs_v, r)
        c = sload(cnt_v, r)
        e_end = s + c
        b0 = (s // _L) * _L
        nch = (e_end - b0 + _L - 1) // _L

        def max_body(k, mc):
            b = b0 + k * _L
            v = score_v[pl.ds(b, _L)]
            g = b + iota
            msk = (g >= s) & (g < e_end) & (v > 0)
            return jnp.maximum(mc, jnp.max(jnp.where(msk, v, NEG), axis=0))

        def soft_iter(it, mc):
            def sum_body(k, acc):
                b = b0 + k * _L
                v = score_v[pl.ds(b, _L)]
                w = winner_v[pl.ds(b, _L)]
                g = b + iota
                msk = (g >= s) & (g < e_end) & (v > 0) & (w == g)
                return acc + jnp.sum(jnp.where(msk, jnp.exp(v - mc), 0.0),
                                     axis=0)

            den = lax.fori_loop(0, nch, sum_body, jnp.float32(0.0))

            def upd_body(k, _):
                b = b0 + k * _L
                v = score_v[pl.ds(b, _L)]
                g = b + iota
                rm = (g >= s) & (g < e_end)
                pos = rm & (v > 0)
                score_v[pl.ds(b, _L)] = jnp.where(
                    pos, jnp.exp(v - mc) / den, jnp.where(rm, 0.0, v))
                return 0

            lax.fori_loop(0, nch, upd_body, 0)
            return (jnp.full((_L,), 1.0, jnp.float32) / den)[0]

        def do_rows(_):
            m0 = lax.fori_loop(0, nch, max_body, NEG)

            def do_soft(_):
                def dict_body(p, _):
                    dn = sload(sdep_v, p)
                    db = (dn // _L) * _L
                    dvec = wtab_v[pl.ds(db, _L)]
                    wtab_v[pl.ds(db, _L)] = jnp.where(iota == dn - db, p, dvec)
                    return 0

                lax.fori_loop(s, e_end, dict_body, 0)

                def wg_body(k, _):
                    b = b0 + k * _L
                    di = sdep_v[pl.ds(b, _L)]
                    winner_v[pl.ds(b, _L)] = plsc.load_gather(wtab_v, [di])
                    return 0

                lax.fori_loop(0, nch, wg_body, 0)
                lax.fori_loop(0, c, soft_iter, m0)
                return 0

            def do_unif(_):
                u = jnp.float32(1.0 / n)

                def ub(k, _):
                    b = b0 + k * _L
                    v = score_v[pl.ds(b, _L)]
                    g = b + iota
                    rm = (g >= s) & (g < e_end)
                    score_v[pl.ds(b, _L)] = jnp.where(rm, u, v)
                    return 0

                lax.fori_loop(0, nch, ub, 0)
                return 0

            lax.cond(m0 > NEG, do_soft, do_unif, 0)
            return 0

        lax.cond(c > 0, do_rows, lambda _: 0, 0)
        return 0

    lax.fori_loop(0, rpw, row_body, 0)

    zero16 = jnp.zeros((_L,), jnp.float32)

    def z_body(i, _):
        for j in range(d // _L):
            agg_v[i, pl.ds(j * _L, _L)] = zero16
        return 0

    lax.fori_loop(0, rpw, z_body, 0)

    k0 = s0 // _CH
    k1 = (s1 + _CH - 1) // _CH

    def gdma(k, buf, s):
        return pltpu.async_copy(
            wx_hbm.at[sdep_v.at[pl.ds(k * _CH, _CH)]], buf, s)

    def issue(k, par):
        lax.cond(par == 0,
                 lambda _: (gdma(k, gbuf.at[0], sem_a), 0)[1],
                 lambda _: (gdma(k, gbuf.at[1], sem_b), 0)[1], 0)

    def acc_chunk(k, buf):
        b = k * _CH

        def e_body(e, _):
            p = b + e

            def acc(_):
                row = sload(sgov_v, p) - r0
                av = sload(score_v, p)
                for j in range(d // _L):
                    cur = agg_v[row, pl.ds(j * _L, _L)]
                    agg_v[row, pl.ds(j * _L, _L)] = (
                        cur + av * buf[e, pl.ds(j * _L, _L)])
                return 0

            lax.cond((p >= s0) & (p < s1), acc, lambda _: 0, 0)
            return 0

        lax.fori_loop(0, _CH, e_body, 0)

    lax.cond(k0 < k1, lambda _: issue(k0, 0) or 0, lambda _: 0, 0)

    def chunk_body(j, _):
        k = k0 + j
        par = j % 2
        lax.cond(k + 1 < k1, lambda _: issue(k + 1, 1 - par) or 0,
                 lambda _: 0, 0)

        def w0(_):
            pltpu.make_async_copy(
                wx_hbm.at[sdep_v.at[pl.ds(k * _CH, _CH)]],
                gbuf.at[0], sem_a).wait()
            acc_chunk(k, gbuf.at[0])
            return 0

        def w1(_):
            pltpu.make_async_copy(
                wx_hbm.at[sdep_v.at[pl.ds(k * _CH, _CH)]],
                gbuf.at[1], sem_b).wait()
            acc_chunk(k, gbuf.at[1])
            return 0

        lax.cond(par == 0, w0, w1, 0)
        return 0

    lax.fori_loop(0, k1 - k0, chunk_body, 0)

    h0_dma.wait()

    def o_body(i, _):
        for j in range(d // _L):
            t = agg_v[i, pl.ds(j * _L, _L)] + hbuf[i, pl.ds(j * _L, _L)]
            hbuf[i, pl.ds(j * _L, _L)] = jnp.where(t >= 0, t, 0.2 * t)
        return 0

    lax.fori_loop(0, rpw, o_body, 0)
    pltpu.async_copy(hbuf, out_hbm.at[outpos_v], sem).wait()


def kernel(_input, dependency_triples, W, a):
    n, d = _input.shape
    m = dependency_triples.shape[0]
    rpw = n // _NW

    deps = dependency_triples[:, 0].astype(jnp.int32)
    govs = dependency_triples[:, 2].astype(jnp.int32)

    mp2 = 1 << max(1, (n - 1).bit_length())
    mesh = plsc.VectorSubcoreMesh(core_axis_name="c", subcore_axis_name="s")
    occ = pl.kernel(
        functools.partial(_occ_kernel, n, m),
        mesh=mesh,
        compiler_params=pltpu.CompilerParams(needs_layout_passes=False),
        out_type=[jax.ShapeDtypeStruct((_NW, n), jnp.int32),
                  jax.ShapeDtypeStruct((_NW, n), jnp.int32),
                  jax.ShapeDtypeStruct((_NW, n), jnp.int32),
                  jax.ShapeDtypeStruct((m,), jnp.int32)],
        scratch_types=[
            pltpu.VMEM((m // _NW + _L,), jnp.int32),
            pltpu.VMEM((m // _NW + _L,), jnp.int32),
            pltpu.VMEM((n,), jnp.int32),
            pltpu.VMEM((n,), jnp.int32),
            pltpu.VMEM((n,), jnp.int32),
            pltpu.VMEM((m // _NW,), jnp.int32),
            pltpu.SemaphoreType.DMA,
            pltpu.SemaphoreType.DMA,
            pltpu.SemaphoreType.DMA,
        ],
    )
    last_tbl, first_tbl, cnt_tbl, rank = occ(deps, govs)
    packed = jnp.max(last_tbl, axis=0)
    h0gov = packed % n
    first_idx = jnp.min(first_tbl, axis=0)
    first_idx = first_idx.at[0].set(-1)
    keys = jnp.argsort(first_idx)
    outpos = jnp.zeros((n,), jnp.int32).at[keys].set(jnp.arange(n, dtype=jnp.int32))

    counts = jnp.sum(cnt_tbl, axis=0)
    csum = jnp.cumsum(counts).astype(jnp.int32)
    row_start = csum - counts
    offs = (row_start[None, :]
            + jnp.cumsum(cnt_tbl, axis=0).astype(jnp.int32) - cnt_tbl)

    scat = pl.kernel(
        functools.partial(_scat_kernel, n, m, mp2),
        mesh=mesh,
        compiler_params=pltpu.CompilerParams(needs_layout_passes=False),
        out_type=jax.ShapeDtypeStruct((m + _EC,), jnp.int32),
        scratch_types=[
            pltpu.VMEM((m // _NW + _L,), jnp.int32),
            pltpu.VMEM((m // _NW + _L,), jnp.int32),
            pltpu.VMEM((m // _NW + _L,), jnp.int32),
            pltpu.VMEM((n + _L,), jnp.int32),
            pltpu.VMEM((m // _NW,), jnp.int32),
            pltpu.VMEM((m // _NW,), jnp.int32),
            pltpu.SemaphoreType.DMA,
            pltpu.SemaphoreType.DMA,
            pltpu.SemaphoreType.DMA,
        ],
    )
    skey = scat(deps, govs, offs, rank)

    a2 = a.reshape(2, d)
    wx, sgd = pl.pallas_call(
        _tc_mm,
        out_shape=[jax.ShapeDtypeStruct((n, d), jnp.float32),
                   jax.ShapeDtypeStruct((n, 2), jnp.float32)],
    )(_input, W, a2)
    sg = sgd[:, 0]
    sd = sgd[:, 1]

    sc = pl.kernel(
        functools.partial(_sc_kernel, n, m, d, rpw, mp2),
        mesh=mesh,
        compiler_params=pltpu.CompilerParams(needs_layout_passes=False),
        out_type=jax.ShapeDtypeStruct((n, d), jnp.float32),
        scratch_types=[
            pltpu.VMEM((m + _EC,), jnp.int32),
            pltpu.VMEM((m + _L,), jnp.int32),
            pltpu.VMEM((m + _L,), jnp.int32),
            pltpu.VMEM((m + _L,), jnp.int32),
            pltpu.VMEM((m + _L,), jnp.float32),
            pltpu.VMEM((mp2,), jnp.int32),
            pltpu.VMEM((mp2,), jnp.float32),
            pltpu.VMEM((mp2,), jnp.float32),
            pltpu.VMEM((rpw + _L,), jnp.int32),
            pltpu.VMEM((rpw + _L,), jnp.int32),
            pltpu.VMEM((rpw,), jnp.int32),
            pltpu.VMEM((rpw,), jnp.int32),
            pltpu.VMEM((2, _CH, d), jnp.float32),
            pltpu.VMEM((rpw, d), jnp.float32),
            pltpu.VMEM((rpw, d), jnp.float32),
            pltpu.SemaphoreType.DMA,
            pltpu.SemaphoreType.DMA,
            pltpu.SemaphoreType.DMA,
            pltpu.SemaphoreType.DMA,
        ],
    )
    return sc(wx, sg, sd, skey, row_start, counts, h0gov, outpos)
